# Initial kernel scaffold; baseline (speedup 1.0000x reference)
import jax
import jax.numpy as jnp
from jax.experimental import pallas as pl


def kernel(edge_src, edge_dst, edge_weight_cutoff, edge_attr, node_feat, Wk0, Wk1, Wk2, Wlogit, Wv0, Wv1, Wv2, Wout):
    raise NotImplementedError("write your pallas kernel here")


# speedup vs baseline: 2.0696x; 2.0696x over previous
import functools
import math

import jax
import jax.numpy as jnp
from jax import lax
from jax.experimental import pallas as pl
from jax.experimental.pallas import tpu as pltpu
from jax.experimental.pallas import tpu_sc as plsc

N = 10000
E = 160000
D = 128
DE = 16
H = 2

NC = 2
NS = 16
NW = NC * NS
PER_W = E // NW
CH = 200
NCH = PER_W // CH
PER_S = E // NS
NCH_S = PER_S // CH

BE = 2000
GB = E // BE
BN = 2000
GN = N // BN

_HI = jax.lax.Precision.HIGHEST


def _f32(*shape):
    return jax.ShapeDtypeStruct(shape, jnp.float32)


def _logit_body(attr_ref, xs_ref, xd_ref, cut_ref, wk0, wk1, wk2, wl2, w_ref):
    x = attr_ref[...]
    x = jax.nn.gelu(jnp.dot(x, wk0[...], precision=_HI))
    x = jax.nn.gelu(jnp.dot(x, wk1[...], precision=_HI))
    k = jnp.dot(x, wk2[...], precision=_HI)
    kk = xs_ref[...] * k
    a = jnp.dot(kk, wl2[...], precision=_HI)
    xd = xd_ref[...]
    l0 = jnp.sum(xd * a[:, :D], axis=1, keepdims=True)
    l1 = jnp.sum(xd * a[:, D:], axis=1, keepdims=True)
    logit = jnp.concatenate([l0, l1], axis=1) * (1.0 / D)
    w01 = cut_ref[...] * jnp.exp(jnp.clip(logit, -60.0, 60.0))
    DH = D // H
    w_ref[...] = jnp.concatenate(
        [jnp.broadcast_to(w01[:, 0:1], (w01.shape[0], DH)),
         jnp.broadcast_to(w01[:, 1:2], (w01.shape[0], DH))], axis=1)


def _tc_logits(edge_attr, xs, xd, cut2, wk0, wk1, wk2, wl2):
    return pl.pallas_call(
        _logit_body,
        grid=(GB,),
        in_specs=[
            pl.BlockSpec((BE, DE), lambda i: (i, 0)),
            pl.BlockSpec((BE, D), lambda i: (i, 0)),
            pl.BlockSpec((BE, D), lambda i: (i, 0)),
            pl.BlockSpec((BE, 1), lambda i: (i, 0)),
            pl.BlockSpec((DE, 64), lambda i: (0, 0)),
            pl.BlockSpec((64, 64), lambda i: (0, 0)),
            pl.BlockSpec((64, D), lambda i: (0, 0)),
            pl.BlockSpec((D, 2 * D), lambda i: (0, 0)),
        ],
        out_specs=pl.BlockSpec((BE, D), lambda i: (i, 0)),
        out_shape=_f32(E, D),
    )(edge_attr, xs, xd, cut2, wk0, wk1, wk2, wl2)


def _edgev_body(attr_ref, xs_ref, w_ref, zd_ref, wv0, wv1, wv2, ev_ref):
    x = attr_ref[...]
    x = jax.nn.gelu(jnp.dot(x, wv0[...], precision=_HI))
    x = jax.nn.gelu(jnp.dot(x, wv1[...], precision=_HI))
    v = jnp.dot(x, wv2[...], precision=_HI)
    v = xs_ref[...] * v
    w = w_ref[...]
    zd = zd_ref[...]
    zd = jnp.where(zd == 0.0, 1.0, zd)
    s = jnp.sqrt(jnp.maximum(w / zd, 0.0))
    ev_ref[...] = v * s


def _tc_edgev(edge_attr, xs, w16, zd, wv0, wv1, wv2):
    return pl.pallas_call(
        _edgev_body,
        grid=(GB,),
        in_specs=[
            pl.BlockSpec((BE, DE), lambda i: (i, 0)),
            pl.BlockSpec((BE, D), lambda i: (i, 0)),
            pl.BlockSpec((BE, D), lambda i: (i, 0)),
            pl.BlockSpec((BE, D), lambda i: (i, 0)),
            pl.BlockSpec((DE, 64), lambda i: (0, 0)),
            pl.BlockSpec((64, 64), lambda i: (0, 0)),
            pl.BlockSpec((64, D), lambda i: (0, 0)),
        ],
        out_specs=pl.BlockSpec((BE, D), lambda i: (i, 0)),
        out_shape=_f32(E, D),
    )(edge_attr, xs, w16, zd, wv0, wv1, wv2)


def _final_body(p0_ref, p1_ref, wout, out_ref):
    acc = p0_ref[...] + p1_ref[...]
    out_ref[...] = jnp.dot(acc, wout[...], precision=_HI)


def _tc_final(p0, p1, wout):
    return pl.pallas_call(
        _final_body,
        grid=(GN,),
        in_specs=[
            pl.BlockSpec((BN, D), lambda i: (i, 0)),
            pl.BlockSpec((BN, D), lambda i: (i, 0)),
            pl.BlockSpec((D, D), lambda i: (0, 0)),
        ],
        out_specs=pl.BlockSpec((BN, D), lambda i: (i, 0)),
        out_shape=_f32(N, D),
    )(p0, p1, wout)


def _build_sc_kernels():
    mesh = plsc.VectorSubcoreMesh(core_axis_name="c", subcore_axis_name="s")

    @functools.partial(
        pl.kernel,
        out_type=(_f32(E, D), _f32(E, D)),
        mesh=mesh,
        scratch_types=[
            pltpu.VMEM((CH,), jnp.int32),
            pltpu.VMEM((CH, D), jnp.float32),
        ],
    )
    def sc_gather_feat(feat_hbm, src_hbm, dst_hbm, xs_hbm, xd_hbm,
                       idx_v, rows_v):
        wid = lax.axis_index("s") * NC + lax.axis_index("c")
        base = wid * PER_W
        for idx_hbm, out_hbm in ((src_hbm, xs_hbm), (dst_hbm, xd_hbm)):
            @pl.loop(0, NCH)
            def _(c, idx_hbm=idx_hbm, out_hbm=out_hbm):
                off = base + c * CH
                pltpu.sync_copy(idx_hbm.at[pl.ds(off, CH)], idx_v)
                pltpu.sync_copy(feat_hbm.at[idx_v], rows_v)
                pltpu.sync_copy(rows_v, out_hbm.at[pl.ds(off, CH)])

    @functools.partial(
        pl.kernel,
        out_type=_f32(N, D),
        mesh=mesh,
        scratch_types=[
            pltpu.VMEM((CH,), jnp.int32),
            pltpu.VMEM((CH, D), jnp.float32),
            pltpu.VMEM_SHARED((N, D), jnp.float32),
        ],
    )
    def sc_scatter_z(w_hbm, dst_hbm, zero_hbm, z_hbm, idx_v, w_v, acc_sh):
        core = lax.axis_index("c")
        sid = lax.axis_index("s")

        @pl.when(core == 0)
        def _():
            @pl.when(sid == 0)
            def _():
                pltpu.sync_copy(zero_hbm, acc_sh)
            plsc.subcore_barrier()
            base = sid * PER_S

            @pl.loop(0, NCH_S)
            def _(c):
                off = base + c * CH
                pltpu.sync_copy(dst_hbm.at[pl.ds(off, CH)], idx_v)
                pltpu.sync_copy(w_hbm.at[pl.ds(off, CH)], w_v)
                pltpu.sync_copy(w_v, acc_sh.at[idx_v], add=True)

            plsc.subcore_barrier()

            @pl.when(sid == 0)
            def _():
                pltpu.sync_copy(acc_sh, z_hbm)

    @functools.partial(
        pl.kernel,
        out_type=_f32(E, D),
        mesh=mesh,
        scratch_types=[
            pltpu.VMEM((CH,), jnp.int32),
            pltpu.VMEM((CH, D), jnp.float32),
        ],
    )
    def sc_gather_z(z_hbm, dst_hbm, zd_hbm, idx_v, rows_v):
        wid = lax.axis_index("s") * NC + lax.axis_index("c")
        base = wid * PER_W

        @pl.loop(0, NCH)
        def _(c):
            off = base + c * CH
            pltpu.sync_copy(dst_hbm.at[pl.ds(off, CH)], idx_v)
            pltpu.sync_copy(z_hbm.at[idx_v], rows_v)
            pltpu.sync_copy(rows_v, zd_hbm.at[pl.ds(off, CH)])

    @functools.partial(
        pl.kernel,
        out_type=(_f32(N, D), _f32(N, D)),
        mesh=mesh,
        scratch_types=[
            pltpu.VMEM((CH,), jnp.int32),
            pltpu.VMEM((CH, D), jnp.float32),
            pltpu.VMEM_SHARED((N, D), jnp.float32),
        ],
    )
    def sc_scatter_out(ev_hbm, dst_hbm, zero_hbm, o0_hbm, o1_hbm,
                       idx_v, ev_v, acc_sh):
        core = lax.axis_index("c")
        sid = lax.axis_index("s")

        @pl.when(sid == 0)
        def _():
            pltpu.sync_copy(zero_hbm, acc_sh)
        plsc.subcore_barrier()
        base = (sid * NC + core) * PER_W

        @pl.loop(0, NCH)
        def _(c):
            off = base + c * CH
            pltpu.sync_copy(dst_hbm.at[pl.ds(off, CH)], idx_v)
            pltpu.sync_copy(ev_hbm.at[pl.ds(off, CH)], ev_v)
            pltpu.sync_copy(ev_v, acc_sh.at[idx_v], add=True)

        plsc.subcore_barrier()

        @pl.when(sid == 0)
        def _():
            @pl.when(core == 0)
            def _():
                pltpu.sync_copy(acc_sh, o0_hbm)

            @pl.when(core == 1)
            def _():
                pltpu.sync_copy(acc_sh, o1_hbm)

    return sc_gather_feat, sc_scatter_z, sc_gather_z, sc_scatter_out


def kernel(edge_src, edge_dst, edge_weight_cutoff, edge_attr, node_feat,
           Wk0, Wk1, Wk2, Wlogit, Wv0, Wv1, Wv2, Wout):
    sc_gather_feat, sc_scatter_z, sc_gather_z, sc_scatter_out = \
        _build_sc_kernels()

    wk0 = Wk0 * (1.0 / math.sqrt(DE))
    wk1 = Wk1 * (1.0 / 8.0)
    wk2 = Wk2 * (1.0 / 8.0)
    wv0 = Wv0 * (1.0 / math.sqrt(DE))
    wv1 = Wv1 * (1.0 / 8.0)
    wv2 = Wv2 * (1.0 / 8.0)
    wl2 = jnp.transpose(Wlogit, (1, 2, 0)).reshape(D, H * D)
    wout = Wout * (1.0 / math.sqrt(D))
    cut2 = edge_weight_cutoff.reshape(E, 1)
    zerosD = jnp.zeros((N, D), jnp.float32)

    xs, xd = sc_gather_feat(node_feat, edge_src, edge_dst)
    w128 = _tc_logits(edge_attr, xs, xd, cut2, wk0, wk1, wk2, wl2)
    z = sc_scatter_z(w128, edge_dst, zerosD)
    zd = sc_gather_z(z, edge_dst)
    ev = _tc_edgev(edge_attr, xs, w128, zd, wv0, wv1, wv2)
    o0, o1 = sc_scatter_out(ev, edge_dst, zerosD)
    return _tc_final(o0, o1, wout)


# speedup vs baseline: 2.9287x; 1.4151x over previous
import functools
import math

import jax
import jax.numpy as jnp
from jax import lax
from jax.experimental import pallas as pl
from jax.experimental.pallas import tpu as pltpu
from jax.experimental.pallas import tpu_sc as plsc

N = 10000
E = 160000
D = 128
DE = 16
H = 2

NC = 2
NS = 16
NW = NC * NS
PER_W = E // NW
CH = 200
NCH = PER_W // CH
PER_S = E // NS
NCH_S = PER_S // CH

BE = 4000
GB = E // BE
BN = 2000
GN = N // BN

def _f32(*shape):
    return jax.ShapeDtypeStruct(shape, jnp.float32)


def _dot3(x, w):
    xh = x.astype(jnp.bfloat16)
    xl = (x - xh.astype(jnp.float32)).astype(jnp.bfloat16)
    wh = w.astype(jnp.bfloat16)
    wl = (w - wh.astype(jnp.float32)).astype(jnp.bfloat16)
    out = jnp.dot(xh, wh, preferred_element_type=jnp.float32)
    out = out + jnp.dot(xh, wl, preferred_element_type=jnp.float32)
    out = out + jnp.dot(xl, wh, preferred_element_type=jnp.float32)
    return out


def _logit_body(attr_ref, xs_ref, xd_ref, cut_ref, wk0, wk1, wk2, wl2, w_ref):
    x = attr_ref[...]
    x = jax.nn.gelu(_dot3(x, wk0[...]))
    x = jax.nn.gelu(_dot3(x, wk1[...]))
    k = _dot3(x, wk2[...])
    kk = xs_ref[...] * k
    a = _dot3(kk, wl2[...])
    xd = xd_ref[...]
    l0 = jnp.sum(xd * a[:, :D], axis=1, keepdims=True)
    l1 = jnp.sum(xd * a[:, D:], axis=1, keepdims=True)
    logit = jnp.concatenate([l0, l1], axis=1) * (1.0 / D)
    w01 = cut_ref[...] * jnp.exp(jnp.clip(logit, -60.0, 60.0))
    DH = D // H
    w_ref[...] = jnp.concatenate(
        [jnp.broadcast_to(w01[:, 0:1], (w01.shape[0], DH)),
         jnp.broadcast_to(w01[:, 1:2], (w01.shape[0], DH))], axis=1)


def _tc_logits(edge_attr, xs, xd, cut2, wk0, wk1, wk2, wl2):
    return pl.pallas_call(
        _logit_body,
        grid=(GB,),
        in_specs=[
            pl.BlockSpec((BE, DE), lambda i: (i, 0)),
            pl.BlockSpec((BE, D), lambda i: (i, 0)),
            pl.BlockSpec((BE, D), lambda i: (i, 0)),
            pl.BlockSpec((BE, 1), lambda i: (i, 0)),
            pl.BlockSpec((DE, 64), lambda i: (0, 0)),
            pl.BlockSpec((64, 64), lambda i: (0, 0)),
            pl.BlockSpec((64, D), lambda i: (0, 0)),
            pl.BlockSpec((D, 2 * D), lambda i: (0, 0)),
        ],
        out_specs=pl.BlockSpec((BE, D), lambda i: (i, 0)),
        out_shape=_f32(E, D),
    )(edge_attr, xs, xd, cut2, wk0, wk1, wk2, wl2)


def _edgev_body(attr_ref, xs_ref, w_ref, zd_ref, wv0, wv1, wv2, ev_ref):
    x = attr_ref[...]
    x = jax.nn.gelu(_dot3(x, wv0[...]))
    x = jax.nn.gelu(_dot3(x, wv1[...]))
    v = _dot3(x, wv2[...])
    v = xs_ref[...] * v
    w = w_ref[...]
    zd = zd_ref[...]
    zd = jnp.where(zd == 0.0, 1.0, zd)
    s = jnp.sqrt(jnp.maximum(w / zd, 0.0))
    ev_ref[...] = v * s


def _tc_edgev(edge_attr, xs, w16, zd, wv0, wv1, wv2):
    return pl.pallas_call(
        _edgev_body,
        grid=(GB,),
        in_specs=[
            pl.BlockSpec((BE, DE), lambda i: (i, 0)),
            pl.BlockSpec((BE, D), lambda i: (i, 0)),
            pl.BlockSpec((BE, D), lambda i: (i, 0)),
            pl.BlockSpec((BE, D), lambda i: (i, 0)),
            pl.BlockSpec((DE, 64), lambda i: (0, 0)),
            pl.BlockSpec((64, 64), lambda i: (0, 0)),
            pl.BlockSpec((64, D), lambda i: (0, 0)),
        ],
        out_specs=pl.BlockSpec((BE, D), lambda i: (i, 0)),
        out_shape=_f32(E, D),
    )(edge_attr, xs, w16, zd, wv0, wv1, wv2)


def _addz_body(z0_ref, z1_ref, z_ref):
    z_ref[...] = z0_ref[...] + z1_ref[...]


def _tc_addz(z0, z1):
    return pl.pallas_call(
        _addz_body,
        grid=(GN,),
        in_specs=[
            pl.BlockSpec((BN, D), lambda i: (i, 0)),
            pl.BlockSpec((BN, D), lambda i: (i, 0)),
        ],
        out_specs=pl.BlockSpec((BN, D), lambda i: (i, 0)),
        out_shape=_f32(N, D),
    )(z0, z1)


def _final_body(p0_ref, p1_ref, wout, out_ref):
    acc = p0_ref[...] + p1_ref[...]
    out_ref[...] = _dot3(acc, wout[...])


def _tc_final(p0, p1, wout):
    return pl.pallas_call(
        _final_body,
        grid=(GN,),
        in_specs=[
            pl.BlockSpec((BN, D), lambda i: (i, 0)),
            pl.BlockSpec((BN, D), lambda i: (i, 0)),
            pl.BlockSpec((D, D), lambda i: (0, 0)),
        ],
        out_specs=pl.BlockSpec((BN, D), lambda i: (i, 0)),
        out_shape=_f32(N, D),
    )(p0, p1, wout)


def _build_sc_kernels():
    mesh = plsc.VectorSubcoreMesh(core_axis_name="c", subcore_axis_name="s")

    @functools.partial(
        pl.kernel,
        out_type=(_f32(E, D), _f32(E, D)),
        mesh=mesh,
        scratch_types=[
            pltpu.VMEM((CH,), jnp.int32),
            pltpu.VMEM((CH, D), jnp.float32),
        ],
    )
    def sc_gather_feat(feat_hbm, src_hbm, dst_hbm, xs_hbm, xd_hbm,
                       idx_v, rows_v):
        wid = lax.axis_index("s") * NC + lax.axis_index("c")
        base = wid * PER_W
        for idx_hbm, out_hbm in ((src_hbm, xs_hbm), (dst_hbm, xd_hbm)):
            @pl.loop(0, NCH)
            def _(c, idx_hbm=idx_hbm, out_hbm=out_hbm):
                off = base + c * CH
                pltpu.sync_copy(idx_hbm.at[pl.ds(off, CH)], idx_v)
                pltpu.sync_copy(feat_hbm.at[idx_v], rows_v)
                pltpu.sync_copy(rows_v, out_hbm.at[pl.ds(off, CH)])

    @functools.partial(
        pl.kernel,
        out_type=(_f32(N, D), _f32(N, D)),
        mesh=mesh,
        scratch_types=[
            pltpu.VMEM((CH,), jnp.int32),
            pltpu.VMEM((CH, D), jnp.float32),
            pltpu.VMEM_SHARED((N, D), jnp.float32),
        ],
    )
    def sc_scatter_z(w_hbm, dst_hbm, zero_hbm, z0_hbm, z1_hbm,
                     idx_v, w_v, acc_sh):
        core = lax.axis_index("c")
        sid = lax.axis_index("s")

        @pl.when(sid == 0)
        def _():
            pltpu.sync_copy(zero_hbm, acc_sh)
        plsc.subcore_barrier()
        base = (sid * NC + core) * PER_W

        @pl.loop(0, NCH)
        def _(c):
            off = base + c * CH
            pltpu.sync_copy(dst_hbm.at[pl.ds(off, CH)], idx_v)
            pltpu.sync_copy(w_hbm.at[pl.ds(off, CH)], w_v)
            pltpu.sync_copy(w_v, acc_sh.at[idx_v], add=True)

        plsc.subcore_barrier()

        @pl.when(sid == 0)
        def _():
            @pl.when(core == 0)
            def _():
                pltpu.sync_copy(acc_sh, z0_hbm)

            @pl.when(core == 1)
            def _():
                pltpu.sync_copy(acc_sh, z1_hbm)

    @functools.partial(
        pl.kernel,
        out_type=_f32(E, D),
        mesh=mesh,
        scratch_types=[
            pltpu.VMEM((CH,), jnp.int32),
            pltpu.VMEM((CH, D), jnp.float32),
        ],
    )
    def sc_gather_z(z_hbm, dst_hbm, zd_hbm, idx_v, rows_v):
        wid = lax.axis_index("s") * NC + lax.axis_index("c")
        base = wid * PER_W

        @pl.loop(0, NCH)
        def _(c):
            off = base + c * CH
            pltpu.sync_copy(dst_hbm.at[pl.ds(off, CH)], idx_v)
            pltpu.sync_copy(z_hbm.at[idx_v], rows_v)
            pltpu.sync_copy(rows_v, zd_hbm.at[pl.ds(off, CH)])

    @functools.partial(
        pl.kernel,
        out_type=(_f32(N, D), _f32(N, D)),
        mesh=mesh,
        scratch_types=[
            pltpu.VMEM((CH,), jnp.int32),
            pltpu.VMEM((CH, D), jnp.float32),
            pltpu.VMEM_SHARED((N, D), jnp.float32),
        ],
    )
    def sc_scatter_out(ev_hbm, dst_hbm, zero_hbm, o0_hbm, o1_hbm,
                       idx_v, ev_v, acc_sh):
        core = lax.axis_index("c")
        sid = lax.axis_index("s")

        @pl.when(sid == 0)
        def _():
            pltpu.sync_copy(zero_hbm, acc_sh)
        plsc.subcore_barrier()
        base = (sid * NC + core) * PER_W

        @pl.loop(0, NCH)
        def _(c):
            off = base + c * CH
            pltpu.sync_copy(dst_hbm.at[pl.ds(off, CH)], idx_v)
            pltpu.sync_copy(ev_hbm.at[pl.ds(off, CH)], ev_v)
            pltpu.sync_copy(ev_v, acc_sh.at[idx_v], add=True)

        plsc.subcore_barrier()

        @pl.when(sid == 0)
        def _():
            @pl.when(core == 0)
            def _():
                pltpu.sync_copy(acc_sh, o0_hbm)

            @pl.when(core == 1)
            def _():
                pltpu.sync_copy(acc_sh, o1_hbm)

    return sc_gather_feat, sc_scatter_z, sc_gather_z, sc_scatter_out


def kernel(edge_src, edge_dst, edge_weight_cutoff, edge_attr, node_feat,
           Wk0, Wk1, Wk2, Wlogit, Wv0, Wv1, Wv2, Wout):
    sc_gather_feat, sc_scatter_z, sc_gather_z, sc_scatter_out = \
        _build_sc_kernels()

    wk0 = Wk0 * (1.0 / math.sqrt(DE))
    wk1 = Wk1 * (1.0 / 8.0)
    wk2 = Wk2 * (1.0 / 8.0)
    wv0 = Wv0 * (1.0 / math.sqrt(DE))
    wv1 = Wv1 * (1.0 / 8.0)
    wv2 = Wv2 * (1.0 / 8.0)
    wl2 = jnp.transpose(Wlogit, (1, 2, 0)).reshape(D, H * D)
    wout = Wout * (1.0 / math.sqrt(D))
    cut2 = edge_weight_cutoff.reshape(E, 1)
    zerosD = jnp.zeros((N, D), jnp.float32)

    xs, xd = sc_gather_feat(node_feat, edge_src, edge_dst)
    w128 = _tc_logits(edge_attr, xs, xd, cut2, wk0, wk1, wk2, wl2)
    zp0, zp1 = sc_scatter_z(w128, edge_dst, zerosD)
    z = _tc_addz(zp0, zp1)
    zd = sc_gather_z(z, edge_dst)
    ev = _tc_edgev(edge_attr, xs, w128, zd, wv0, wv1, wv2)
    o0, o1 = sc_scatter_out(ev, edge_dst, zerosD)
    return _tc_final(o0, o1, wout)


# speedup vs baseline: 3.6144x; 1.2341x over previous
import functools
import math

import jax
import jax.numpy as jnp
from jax import lax
from jax.experimental import pallas as pl
from jax.experimental.pallas import tpu as pltpu
from jax.experimental.pallas import tpu_sc as plsc

N = 10000
E = 160000
D = 128
DE = 16
H = 2

NC = 2
NS = 16
NW = NC * NS
PER_W = E // NW
CH = 200
NCH = PER_W // CH
PER_S = E // NS
NCH_S = PER_S // CH

BE = 4000
GB = E // BE
BN = 2000
GN = N // BN

def _f32(*shape):
    return jax.ShapeDtypeStruct(shape, jnp.float32)


def _dot3(x, w):
    xh = x.astype(jnp.bfloat16)
    xl = (x - xh.astype(jnp.float32)).astype(jnp.bfloat16)
    wh = w.astype(jnp.bfloat16)
    wl = (w - wh.astype(jnp.float32)).astype(jnp.bfloat16)
    out = jnp.dot(xh, wh, preferred_element_type=jnp.float32)
    out = out + jnp.dot(xh, wl, preferred_element_type=jnp.float32)
    out = out + jnp.dot(xl, wh, preferred_element_type=jnp.float32)
    return out


def _logit_body(attr_ref, xs_ref, xd_ref, cut_ref, wk0, wk1, wk2, wl2, hsel,
                w_ref):
    x = attr_ref[...]
    x = jax.nn.gelu(_dot3(x, wk0[...]))
    x = jax.nn.gelu(_dot3(x, wk1[...]))
    k = _dot3(x, wk2[...])
    kk = xs_ref[...] * k
    a = _dot3(kk, wl2[...])
    xd = xd_ref[...]
    m = jnp.concatenate([xd, xd], axis=1) * a
    logit = jnp.dot(m.astype(jnp.bfloat16), hsel[...].astype(jnp.bfloat16),
                    preferred_element_type=jnp.float32) * (1.0 / D)
    w01 = cut_ref[...] * jnp.exp(jnp.clip(logit, -60.0, 60.0))
    DH = D // H
    w_ref[...] = jnp.concatenate(
        [jnp.broadcast_to(w01[:, 0:1], (w01.shape[0], DH)),
         jnp.broadcast_to(w01[:, 1:2], (w01.shape[0], DH))], axis=1)


def _tc_logits(edge_attr, xs, xd, cut2, wk0, wk1, wk2, wl2, hsel):
    return pl.pallas_call(
        _logit_body,
        grid=(GB,),
        in_specs=[
            pl.BlockSpec((BE, DE), lambda i: (i, 0)),
            pl.BlockSpec((BE, D), lambda i: (i, 0)),
            pl.BlockSpec((BE, D), lambda i: (i, 0)),
            pl.BlockSpec((BE, 1), lambda i: (i, 0)),
            pl.BlockSpec((DE, 64), lambda i: (0, 0)),
            pl.BlockSpec((64, 64), lambda i: (0, 0)),
            pl.BlockSpec((64, D), lambda i: (0, 0)),
            pl.BlockSpec((D, 2 * D), lambda i: (0, 0)),
            pl.BlockSpec((2 * D, H), lambda i: (0, 0)),
        ],
        out_specs=pl.BlockSpec((BE, D), lambda i: (i, 0)),
        out_shape=_f32(E, D),
    )(edge_attr, xs, xd, cut2, wk0, wk1, wk2, wl2, hsel)


def _edgev_body(attr_ref, xs_ref, w_ref, zd_ref, wv0, wv1, wv2, ev_ref):
    x = attr_ref[...]
    x = jax.nn.gelu(_dot3(x, wv0[...]))
    x = jax.nn.gelu(_dot3(x, wv1[...]))
    v = _dot3(x, wv2[...])
    v = xs_ref[...] * v
    DH = D // H
    n = v.shape[0]
    w01 = jnp.concatenate([w_ref[:, 0:1], w_ref[:, DH:DH + 1]], axis=1)
    zd01 = jnp.concatenate([zd_ref[:, 0:1], zd_ref[:, DH:DH + 1]], axis=1)
    zd01 = jnp.where(zd01 == 0.0, 1.0, zd01)
    s01 = jnp.sqrt(jnp.maximum(w01 / zd01, 0.0))
    s = jnp.concatenate([jnp.broadcast_to(s01[:, 0:1], (n, DH)),
                         jnp.broadcast_to(s01[:, 1:2], (n, DH))], axis=1)
    ev_ref[...] = v * s


def _tc_edgev(edge_attr, xs, w16, zd, wv0, wv1, wv2):
    return pl.pallas_call(
        _edgev_body,
        grid=(GB,),
        in_specs=[
            pl.BlockSpec((BE, DE), lambda i: (i, 0)),
            pl.BlockSpec((BE, D), lambda i: (i, 0)),
            pl.BlockSpec((BE, D), lambda i: (i, 0)),
            pl.BlockSpec((BE, D), lambda i: (i, 0)),
            pl.BlockSpec((DE, 64), lambda i: (0, 0)),
            pl.BlockSpec((64, 64), lambda i: (0, 0)),
            pl.BlockSpec((64, D), lambda i: (0, 0)),
        ],
        out_specs=pl.BlockSpec((BE, D), lambda i: (i, 0)),
        out_shape=_f32(E, D),
    )(edge_attr, xs, w16, zd, wv0, wv1, wv2)


def _addz_body(z0_ref, z1_ref, z_ref):
    z_ref[...] = z0_ref[...] + z1_ref[...]


def _tc_addz(z0, z1):
    return pl.pallas_call(
        _addz_body,
        grid=(GN,),
        in_specs=[
            pl.BlockSpec((BN, D), lambda i: (i, 0)),
            pl.BlockSpec((BN, D), lambda i: (i, 0)),
        ],
        out_specs=pl.BlockSpec((BN, D), lambda i: (i, 0)),
        out_shape=_f32(N, D),
    )(z0, z1)


def _final_body(p0_ref, p1_ref, wout, out_ref):
    acc = p0_ref[...] + p1_ref[...]
    out_ref[...] = _dot3(acc, wout[...])


def _tc_final(p0, p1, wout):
    return pl.pallas_call(
        _final_body,
        grid=(GN,),
        in_specs=[
            pl.BlockSpec((BN, D), lambda i: (i, 0)),
            pl.BlockSpec((BN, D), lambda i: (i, 0)),
            pl.BlockSpec((D, D), lambda i: (0, 0)),
        ],
        out_specs=pl.BlockSpec((BN, D), lambda i: (i, 0)),
        out_shape=_f32(N, D),
    )(p0, p1, wout)


def _build_sc_kernels():
    mesh = plsc.VectorSubcoreMesh(core_axis_name="c", subcore_axis_name="s")

    @functools.partial(
        pl.kernel,
        out_type=(_f32(E, D), _f32(E, D)),
        mesh=mesh,
        scratch_types=[
            pltpu.VMEM((CH,), jnp.int32),
            pltpu.VMEM((CH, D), jnp.float32),
        ],
    )
    def sc_gather_feat(feat_hbm, src_hbm, dst_hbm, xs_hbm, xd_hbm,
                       idx_v, rows_v):
        wid = lax.axis_index("s") * NC + lax.axis_index("c")
        base = wid * PER_W
        for idx_hbm, out_hbm in ((src_hbm, xs_hbm), (dst_hbm, xd_hbm)):
            @pl.loop(0, NCH)
            def _(c, idx_hbm=idx_hbm, out_hbm=out_hbm):
                off = base + c * CH
                pltpu.sync_copy(idx_hbm.at[pl.ds(off, CH)], idx_v)
                pltpu.sync_copy(feat_hbm.at[idx_v], rows_v)
                pltpu.sync_copy(rows_v, out_hbm.at[pl.ds(off, CH)])

    @functools.partial(
        pl.kernel,
        out_type=(_f32(N, D), _f32(N, D)),
        mesh=mesh,
        scratch_types=[
            pltpu.VMEM((CH,), jnp.int32),
            pltpu.VMEM((CH, D), jnp.float32),
            pltpu.VMEM_SHARED((N, D), jnp.float32),
        ],
    )
    def sc_scatter_z(w_hbm, dst_hbm, zero_hbm, z0_hbm, z1_hbm,
                     idx_v, w_v, acc_sh):
        core = lax.axis_index("c")
        sid = lax.axis_index("s")

        @pl.when(sid == 0)
        def _():
            pltpu.sync_copy(zero_hbm, acc_sh)
        plsc.subcore_barrier()
        base = (sid * NC + core) * PER_W

        @pl.loop(0, NCH)
        def _(c):
            off = base + c * CH
            pltpu.sync_copy(dst_hbm.at[pl.ds(off, CH)], idx_v)
            pltpu.sync_copy(w_hbm.at[pl.ds(off, CH)], w_v)
            pltpu.sync_copy(w_v, acc_sh.at[idx_v], add=True)

        plsc.subcore_barrier()

        @pl.when(sid == 0)
        def _():
            @pl.when(core == 0)
            def _():
                pltpu.sync_copy(acc_sh, z0_hbm)

            @pl.when(core == 1)
            def _():
                pltpu.sync_copy(acc_sh, z1_hbm)

    @functools.partial(
        pl.kernel,
        out_type=_f32(E, D),
        mesh=mesh,
        scratch_types=[
            pltpu.VMEM((CH,), jnp.int32),
            pltpu.VMEM((CH, D), jnp.float32),
        ],
    )
    def sc_gather_z(z_hbm, dst_hbm, zd_hbm, idx_v, rows_v):
        wid = lax.axis_index("s") * NC + lax.axis_index("c")
        base = wid * PER_W

        @pl.loop(0, NCH)
        def _(c):
            off = base + c * CH
            pltpu.sync_copy(dst_hbm.at[pl.ds(off, CH)], idx_v)
            pltpu.sync_copy(z_hbm.at[idx_v], rows_v)
            pltpu.sync_copy(rows_v, zd_hbm.at[pl.ds(off, CH)])

    @functools.partial(
        pl.kernel,
        out_type=(_f32(N, D), _f32(N, D)),
        mesh=mesh,
        scratch_types=[
            pltpu.VMEM((CH,), jnp.int32),
            pltpu.VMEM((CH, D), jnp.float32),
            pltpu.VMEM_SHARED((N, D), jnp.float32),
        ],
    )
    def sc_scatter_out(ev_hbm, dst_hbm, zero_hbm, o0_hbm, o1_hbm,
                       idx_v, ev_v, acc_sh):
        core = lax.axis_index("c")
        sid = lax.axis_index("s")

        @pl.when(sid == 0)
        def _():
            pltpu.sync_copy(zero_hbm, acc_sh)
        plsc.subcore_barrier()
        base = (sid * NC + core) * PER_W

        @pl.loop(0, NCH)
        def _(c):
            off = base + c * CH
            pltpu.sync_copy(dst_hbm.at[pl.ds(off, CH)], idx_v)
            pltpu.sync_copy(ev_hbm.at[pl.ds(off, CH)], ev_v)
            pltpu.sync_copy(ev_v, acc_sh.at[idx_v], add=True)

        plsc.subcore_barrier()

        @pl.when(sid == 0)
        def _():
            @pl.when(core == 0)
            def _():
                pltpu.sync_copy(acc_sh, o0_hbm)

            @pl.when(core == 1)
            def _():
                pltpu.sync_copy(acc_sh, o1_hbm)

    return sc_gather_feat, sc_scatter_z, sc_gather_z, sc_scatter_out


def kernel(edge_src, edge_dst, edge_weight_cutoff, edge_attr, node_feat,
           Wk0, Wk1, Wk2, Wlogit, Wv0, Wv1, Wv2, Wout):
    sc_gather_feat, sc_scatter_z, sc_gather_z, sc_scatter_out = \
        _build_sc_kernels()

    wk0 = Wk0 * (1.0 / math.sqrt(DE))
    wk1 = Wk1 * (1.0 / 8.0)
    wk2 = Wk2 * (1.0 / 8.0)
    wv0 = Wv0 * (1.0 / math.sqrt(DE))
    wv1 = Wv1 * (1.0 / 8.0)
    wv2 = Wv2 * (1.0 / 8.0)
    wl2 = jnp.transpose(Wlogit, (1, 2, 0)).reshape(D, H * D)
    wout = Wout * (1.0 / math.sqrt(D))
    cut2 = edge_weight_cutoff.reshape(E, 1)
    eye = jnp.eye(H, dtype=jnp.float32)
    hsel = jnp.repeat(eye, D, axis=0).reshape(H * D, H)
    zerosD = jnp.zeros((N, D), jnp.float32)

    xs, xd = sc_gather_feat(node_feat, edge_src, edge_dst)
    w128 = _tc_logits(edge_attr, xs, xd, cut2, wk0, wk1, wk2, wl2, hsel)
    zp0, zp1 = sc_scatter_z(w128, edge_dst, zerosD)
    z = _tc_addz(zp0, zp1)
    zd = sc_gather_z(z, edge_dst)
    ev = _tc_edgev(edge_attr, xs, w128, zd, wv0, wv1, wv2)
    o0, o1 = sc_scatter_out(ev, edge_dst, zerosD)
    return _tc_final(o0, o1, wout)


# speedup vs baseline: 3.9170x; 1.0837x over previous
import functools
import math

import jax
import jax.numpy as jnp
from jax import lax
from jax.experimental import pallas as pl
from jax.experimental.pallas import tpu as pltpu
from jax.experimental.pallas import tpu_sc as plsc

N = 10000
E = 160000
D = 128
DE = 16
H = 2
DH = D // H

SLICES = 5
ES = E // SLICES

NC = 2
NS = 16
NW = NC * NS
PW = ES // NW
CH = 200
NCH = PW // CH

BE = 4000
GBS = ES // BE
BN = 2000
GN = N // BN


def _f32(*shape):
    return jax.ShapeDtypeStruct(shape, jnp.float32)


def _dot3(x, w):
    xh = x.astype(jnp.bfloat16)
    xl = (x - xh.astype(jnp.float32)).astype(jnp.bfloat16)
    wh = w.astype(jnp.bfloat16)
    wl = (w - wh.astype(jnp.float32)).astype(jnp.bfloat16)
    out = jnp.dot(xh, wh, preferred_element_type=jnp.float32)
    out = out + jnp.dot(xh, wl, preferred_element_type=jnp.float32)
    out = out + jnp.dot(xl, wh, preferred_element_type=jnp.float32)
    return out


def _logit_body(attr_ref, xs_ref, xd_ref, cut_ref, wk0, wk1, wk2, wl2, hsel,
                w_ref):
    x = attr_ref[...]
    x = jax.nn.gelu(_dot3(x, wk0[...]))
    x = jax.nn.gelu(_dot3(x, wk1[...]))
    k = _dot3(x, wk2[...])
    kk = xs_ref[...] * k
    a = _dot3(kk, wl2[...])
    xd = xd_ref[...]
    m = jnp.concatenate([xd, xd], axis=1) * a
    logit = jnp.dot(m.astype(jnp.bfloat16), hsel[...].astype(jnp.bfloat16),
                    preferred_element_type=jnp.float32) * (1.0 / D)
    w01 = cut_ref[:, 0:1] * jnp.exp(jnp.clip(logit, -60.0, 60.0))
    w_ref[...] = jnp.concatenate(
        [jnp.broadcast_to(w01[:, 0:1], (w01.shape[0], DH)),
         jnp.broadcast_to(w01[:, 1:2], (w01.shape[0], DH))], axis=1)


def _tc_logits(s, edge_attr, xs, xd, cutf, wk0, wk1, wk2, wl2, hsel):
    base = s * GBS
    return pl.pallas_call(
        _logit_body,
        grid=(GBS,),
        in_specs=[
            pl.BlockSpec((BE, DE), lambda i: (base + i, 0)),
            pl.BlockSpec((BE, D), lambda i: (i, 0)),
            pl.BlockSpec((BE, D), lambda i: (i, 0)),
            pl.BlockSpec((BE, D), lambda i: (base + i, 0)),
            pl.BlockSpec((DE, 64), lambda i: (0, 0)),
            pl.BlockSpec((64, 64), lambda i: (0, 0)),
            pl.BlockSpec((64, D), lambda i: (0, 0)),
            pl.BlockSpec((D, 2 * D), lambda i: (0, 0)),
            pl.BlockSpec((2 * D, H), lambda i: (0, 0)),
        ],
        out_specs=pl.BlockSpec((BE, D), lambda i: (i, 0)),
        out_shape=_f32(ES, D),
    )(edge_attr, xs, xd, cutf, wk0, wk1, wk2, wl2, hsel)


def _edgev_body(attr_ref, xs_ref, w_ref, zd_ref, wv0, wv1, wv2, ev_ref):
    x = attr_ref[...]
    x = jax.nn.gelu(_dot3(x, wv0[...]))
    x = jax.nn.gelu(_dot3(x, wv1[...]))
    v = _dot3(x, wv2[...])
    v = xs_ref[...] * v
    n = v.shape[0]
    w01 = jnp.concatenate([w_ref[:, 0:1], w_ref[:, DH:DH + 1]], axis=1)
    zd01 = jnp.concatenate([zd_ref[:, 0:1], zd_ref[:, DH:DH + 1]], axis=1)
    zd01 = jnp.where(zd01 == 0.0, 1.0, zd01)
    s01 = jnp.sqrt(jnp.maximum(w01 / zd01, 0.0))
    s = jnp.concatenate([jnp.broadcast_to(s01[:, 0:1], (n, DH)),
                         jnp.broadcast_to(s01[:, 1:2], (n, DH))], axis=1)
    ev_ref[...] = v * s


def _tc_edgev(s, edge_attr, xs, w, zd, wv0, wv1, wv2):
    base = s * GBS
    return pl.pallas_call(
        _edgev_body,
        grid=(GBS,),
        in_specs=[
            pl.BlockSpec((BE, DE), lambda i: (base + i, 0)),
            pl.BlockSpec((BE, D), lambda i: (i, 0)),
            pl.BlockSpec((BE, D), lambda i: (i, 0)),
            pl.BlockSpec((BE, D), lambda i: (i, 0)),
            pl.BlockSpec((DE, 64), lambda i: (0, 0)),
            pl.BlockSpec((64, 64), lambda i: (0, 0)),
            pl.BlockSpec((64, D), lambda i: (0, 0)),
        ],
        out_specs=pl.BlockSpec((BE, D), lambda i: (i, 0)),
        out_shape=_f32(ES, D),
    )(edge_attr, xs, w, zd, wv0, wv1, wv2)


def _addz_body(z0_ref, z1_ref, z_ref):
    z_ref[...] = z0_ref[...] + z1_ref[...]


def _tc_addz(z0, z1):
    return pl.pallas_call(
        _addz_body,
        grid=(GN,),
        in_specs=[
            pl.BlockSpec((BN, D), lambda i: (i, 0)),
            pl.BlockSpec((BN, D), lambda i: (i, 0)),
        ],
        out_specs=pl.BlockSpec((BN, D), lambda i: (i, 0)),
        out_shape=_f32(N, D),
    )(z0, z1)


def _final_body(*refs):
    parts = refs[:-2]
    wout = refs[-2]
    out_ref = refs[-1]
    acc = parts[0][...]
    for p in parts[1:]:
        acc = acc + p[...]
    out_ref[...] = _dot3(acc, wout[...])


def _tc_final(parts, wout):
    return pl.pallas_call(
        _final_body,
        grid=(GN,),
        in_specs=[pl.BlockSpec((BN, D), lambda i: (i, 0)) for _ in parts]
        + [pl.BlockSpec((D, D), lambda i: (0, 0))],
        out_specs=pl.BlockSpec((BN, D), lambda i: (i, 0)),
        out_shape=_f32(N, D),
    )(*parts, wout)


def _build_sc_kernels():
    mesh = plsc.VectorSubcoreMesh(core_axis_name="c", subcore_axis_name="s")

    def make_gather_feat(s):
        @functools.partial(
            pl.kernel,
            out_type=(_f32(ES, D), _f32(ES, D)),
            mesh=mesh,
            scratch_types=[
                pltpu.VMEM((CH,), jnp.int32),
                pltpu.VMEM((CH, D), jnp.float32),
            ],
        )
        def sc_gather_feat(feat_hbm, src_hbm, dst_hbm, xs_hbm, xd_hbm,
                           idx_v, rows_v):
            wid = lax.axis_index("s") * NC + lax.axis_index("c")
            lbase = wid * PW
            gbase = s * ES + wid * PW
            for idx_hbm, out_hbm in ((src_hbm, xs_hbm), (dst_hbm, xd_hbm)):
                @pl.loop(0, NCH)
                def _(c, idx_hbm=idx_hbm, out_hbm=out_hbm):
                    pltpu.sync_copy(idx_hbm.at[pl.ds(gbase + c * CH, CH)],
                                    idx_v)
                    pltpu.sync_copy(feat_hbm.at[idx_v], rows_v)
                    pltpu.sync_copy(rows_v, out_hbm.at[pl.ds(lbase + c * CH,
                                                             CH)])

        return sc_gather_feat

    @functools.partial(
        pl.kernel,
        out_type=(_f32(N, D), _f32(N, D)),
        mesh=mesh,
        scratch_types=[
            pltpu.VMEM((CH,), jnp.int32),
            pltpu.VMEM((CH, D), jnp.float32),
            pltpu.VMEM_SHARED((N, D), jnp.float32),
        ],
    )
    def sc_scatter_z(w0, w1, w2, w3, w4, dst_hbm, zero_hbm, z0_hbm, z1_hbm,
                     idx_v, w_v, acc_sh):
        core = lax.axis_index("c")
        sid = lax.axis_index("s")
        ws = (w0, w1, w2, w3, w4)

        @pl.when(sid == 0)
        def _():
            pltpu.sync_copy(zero_hbm, acc_sh)
        plsc.subcore_barrier()
        wid = sid * NC + core
        for s in range(SLICES):
            @pl.loop(0, NCH)
            def _(c, s=s, w_hbm=ws[s]):
                loff = wid * PW + c * CH
                pltpu.sync_copy(dst_hbm.at[pl.ds(s * ES + loff, CH)], idx_v)
                pltpu.sync_copy(w_hbm.at[pl.ds(loff, CH)], w_v)
                pltpu.sync_copy(w_v, acc_sh.at[idx_v], add=True)

        plsc.subcore_barrier()

        @pl.when(sid == 0)
        def _():
            @pl.when(core == 0)
            def _():
                pltpu.sync_copy(acc_sh, z0_hbm)

            @pl.when(core == 1)
            def _():
                pltpu.sync_copy(acc_sh, z1_hbm)

    def make_gather_z(s):
        @functools.partial(
            pl.kernel,
            out_type=_f32(ES, D),
            mesh=mesh,
            scratch_types=[
                pltpu.VMEM((CH,), jnp.int32),
                pltpu.VMEM((CH, D), jnp.float32),
            ],
        )
        def sc_gather_z(z_hbm, dst_hbm, zd_hbm, idx_v, rows_v):
            wid = lax.axis_index("s") * NC + lax.axis_index("c")
            lbase = wid * PW
            gbase = s * ES + wid * PW

            @pl.loop(0, NCH)
            def _(c):
                pltpu.sync_copy(dst_hbm.at[pl.ds(gbase + c * CH, CH)], idx_v)
                pltpu.sync_copy(z_hbm.at[idx_v], rows_v)
                pltpu.sync_copy(rows_v, zd_hbm.at[pl.ds(lbase + c * CH, CH)])

        return sc_gather_z

    def make_scatter_out(s):
        @functools.partial(
            pl.kernel,
            out_type=(_f32(N, D), _f32(N, D)),
            mesh=mesh,
            scratch_types=[
                pltpu.VMEM((CH,), jnp.int32),
                pltpu.VMEM((CH, D), jnp.float32),
                pltpu.VMEM_SHARED((N, D), jnp.float32),
            ],
        )
        def sc_scatter_out(ev_hbm, dst_hbm, zero_hbm, o0_hbm, o1_hbm,
                           idx_v, ev_v, acc_sh):
            core = lax.axis_index("c")
            sid = lax.axis_index("s")

            @pl.when(sid == 0)
            def _():
                pltpu.sync_copy(zero_hbm, acc_sh)
            plsc.subcore_barrier()
            wid = sid * NC + core
            lbase = wid * PW
            gbase = s * ES + wid * PW

            @pl.loop(0, NCH)
            def _(c):
                pltpu.sync_copy(dst_hbm.at[pl.ds(gbase + c * CH, CH)], idx_v)
                pltpu.sync_copy(ev_hbm.at[pl.ds(lbase + c * CH, CH)], ev_v)
                pltpu.sync_copy(ev_v, acc_sh.at[idx_v], add=True)

            plsc.subcore_barrier()

            @pl.when(sid == 0)
            def _():
                @pl.when(core == 0)
                def _():
                    pltpu.sync_copy(acc_sh, o0_hbm)

                @pl.when(core == 1)
                def _():
                    pltpu.sync_copy(acc_sh, o1_hbm)

        return sc_scatter_out

    return make_gather_feat, sc_scatter_z, make_gather_z, make_scatter_out


def kernel(edge_src, edge_dst, edge_weight_cutoff, edge_attr, node_feat,
           Wk0, Wk1, Wk2, Wlogit, Wv0, Wv1, Wv2, Wout):
    make_gather_feat, sc_scatter_z, make_gather_z, make_scatter_out = \
        _build_sc_kernels()

    wk0 = Wk0 * (1.0 / math.sqrt(DE))
    wk1 = Wk1 * (1.0 / 8.0)
    wk2 = Wk2 * (1.0 / 8.0)
    wv0 = Wv0 * (1.0 / math.sqrt(DE))
    wv1 = Wv1 * (1.0 / 8.0)
    wv2 = Wv2 * (1.0 / 8.0)
    wl2 = jnp.transpose(Wlogit, (1, 2, 0)).reshape(D, H * D)
    wout = Wout * (1.0 / math.sqrt(D))
    cutf = jnp.broadcast_to(edge_weight_cutoff[:, None], (E, D))
    eye = jnp.eye(H, dtype=jnp.float32)
    hsel = jnp.repeat(eye, D, axis=0).reshape(H * D, H)
    zerosD = jnp.zeros((N, D), jnp.float32)

    xs, xd, w = [], [], []
    for s in range(SLICES):
        a, b = make_gather_feat(s)(node_feat, edge_src, edge_dst)
        xs.append(a)
        xd.append(b)
    for s in range(SLICES):
        w.append(_tc_logits(s, edge_attr, xs[s], xd[s], cutf,
                            wk0, wk1, wk2, wl2, hsel))
    zp0, zp1 = sc_scatter_z(*w, edge_dst, zerosD)
    z = _tc_addz(zp0, zp1)
    parts = []
    for s in range(SLICES):
        zd = make_gather_z(s)(z, edge_dst)
        ev = _tc_edgev(s, edge_attr, xs[s], w[s], zd, wv0, wv1, wv2)
        o0, o1 = make_scatter_out(s)(ev, edge_dst, zerosD)
        parts.extend([o0, o1])
    return _tc_final(parts, wout)


# speedup vs baseline: 4.4973x; 1.1481x over previous
import functools
import math

import jax
import jax.numpy as jnp
from jax import lax
from jax.experimental import pallas as pl
from jax.experimental.pallas import tpu as pltpu
from jax.experimental.pallas import tpu_sc as plsc

N = 10000
E = 160000
D = 128
DE = 16
H = 2
DH = D // H

SLICES = 5
ES = E // SLICES

NC = 2
NS = 16
NW = NC * NS
PW = ES // NW
CH = 200
NCH = PW // CH

BE = 4000
GBS = ES // BE
BN = 2000
GN = N // BN


def _f32(*shape):
    return jax.ShapeDtypeStruct(shape, jnp.float32)


def _dot3(x, w):
    xh = x.astype(jnp.bfloat16)
    xl = (x - xh.astype(jnp.float32)).astype(jnp.bfloat16)
    wh = w.astype(jnp.bfloat16)
    wl = (w - wh.astype(jnp.float32)).astype(jnp.bfloat16)
    out = jnp.dot(xh, wh, preferred_element_type=jnp.float32)
    out = out + jnp.dot(xh, wl, preferred_element_type=jnp.float32)
    out = out + jnp.dot(xl, wh, preferred_element_type=jnp.float32)
    return out


def _dot1(x, w):
    return jnp.dot(x.astype(jnp.bfloat16), w.astype(jnp.bfloat16),
                   preferred_element_type=jnp.float32)


def _logit_body(attr_ref, xs_ref, xd_ref, cut_ref, wk0, wk1, wk2, wl2, hsel,
                w_ref):
    x = attr_ref[...]
    x = jax.nn.gelu(_dot1(x, wk0[...]))
    x = jax.nn.gelu(_dot1(x, wk1[...]))
    k = _dot1(x, wk2[...])
    kk = xs_ref[...] * k
    a = _dot1(kk, wl2[...])
    xd = xd_ref[...]
    m = jnp.concatenate([xd, xd], axis=1) * a
    logit = jnp.dot(m.astype(jnp.bfloat16), hsel[...].astype(jnp.bfloat16),
                    preferred_element_type=jnp.float32) * (1.0 / D)
    w01 = cut_ref[:, 0:1] * jnp.exp(jnp.clip(logit, -60.0, 60.0))
    w_ref[...] = jnp.concatenate(
        [jnp.broadcast_to(w01[:, 0:1], (w01.shape[0], DH)),
         jnp.broadcast_to(w01[:, 1:2], (w01.shape[0], DH))], axis=1)


def _tc_logits(s, edge_attr, xs, xd, cutf, wk0, wk1, wk2, wl2, hsel):
    base = s * GBS
    return pl.pallas_call(
        _logit_body,
        grid=(GBS,),
        in_specs=[
            pl.BlockSpec((BE, DE), lambda i: (base + i, 0)),
            pl.BlockSpec((BE, D), lambda i: (i, 0)),
            pl.BlockSpec((BE, D), lambda i: (i, 0)),
            pl.BlockSpec((BE, D), lambda i: (base + i, 0)),
            pl.BlockSpec((DE, 64), lambda i: (0, 0)),
            pl.BlockSpec((64, 64), lambda i: (0, 0)),
            pl.BlockSpec((64, D), lambda i: (0, 0)),
            pl.BlockSpec((D, 2 * D), lambda i: (0, 0)),
            pl.BlockSpec((2 * D, H), lambda i: (0, 0)),
        ],
        out_specs=pl.BlockSpec((BE, D), lambda i: (i, 0)),
        out_shape=_f32(ES, D),
    )(edge_attr, xs, xd, cutf, wk0, wk1, wk2, wl2, hsel)


def _edgev_body(attr_ref, xs_ref, w_ref, zd_ref, wv0, wv1, wv2, ev_ref):
    x = attr_ref[...]
    x = jax.nn.gelu(_dot1(x, wv0[...]))
    x = jax.nn.gelu(_dot1(x, wv1[...]))
    v = _dot1(x, wv2[...])
    v = xs_ref[...] * v
    n = v.shape[0]
    w01 = jnp.concatenate([w_ref[:, 0:1], w_ref[:, DH:DH + 1]], axis=1)
    zd01 = jnp.concatenate([zd_ref[:, 0:1], zd_ref[:, DH:DH + 1]], axis=1)
    zd01 = jnp.where(zd01 == 0.0, 1.0, zd01)
    s01 = jnp.sqrt(jnp.maximum(w01 / zd01, 0.0))
    s = jnp.concatenate([jnp.broadcast_to(s01[:, 0:1], (n, DH)),
                         jnp.broadcast_to(s01[:, 1:2], (n, DH))], axis=1)
    ev_ref[...] = v * s


def _tc_edgev(s, edge_attr, xs, w, zd, wv0, wv1, wv2):
    base = s * GBS
    return pl.pallas_call(
        _edgev_body,
        grid=(GBS,),
        in_specs=[
            pl.BlockSpec((BE, DE), lambda i: (base + i, 0)),
            pl.BlockSpec((BE, D), lambda i: (i, 0)),
            pl.BlockSpec((BE, D), lambda i: (i, 0)),
            pl.BlockSpec((BE, D), lambda i: (i, 0)),
            pl.BlockSpec((DE, 64), lambda i: (0, 0)),
            pl.BlockSpec((64, 64), lambda i: (0, 0)),
            pl.BlockSpec((64, D), lambda i: (0, 0)),
        ],
        out_specs=pl.BlockSpec((BE, D), lambda i: (i, 0)),
        out_shape=_f32(ES, D),
    )(edge_attr, xs, w, zd, wv0, wv1, wv2)


def _addn_body(*refs):
    out_ref = refs[-1]
    acc = refs[0][...]
    for p in refs[1:-1]:
        acc = acc + p[...]
    out_ref[...] = acc


def _tc_addn(parts):
    return pl.pallas_call(
        _addn_body,
        grid=(GN,),
        in_specs=[pl.BlockSpec((BN, D), lambda i: (i, 0)) for _ in parts],
        out_specs=pl.BlockSpec((BN, D), lambda i: (i, 0)),
        out_shape=_f32(N, D),
    )(*parts)


def _final_body(*refs):
    parts = refs[:-2]
    wout = refs[-2]
    out_ref = refs[-1]
    acc = parts[0][...]
    for p in parts[1:]:
        acc = acc + p[...]
    out_ref[...] = _dot3(acc, wout[...])


def _tc_final(parts, wout):
    return pl.pallas_call(
        _final_body,
        grid=(GN,),
        in_specs=[pl.BlockSpec((BN, D), lambda i: (i, 0)) for _ in parts]
        + [pl.BlockSpec((D, D), lambda i: (0, 0))],
        out_specs=pl.BlockSpec((BN, D), lambda i: (i, 0)),
        out_shape=_f32(N, D),
    )(*parts, wout)


def _build_sc_kernels():
    mesh = plsc.VectorSubcoreMesh(core_axis_name="c", subcore_axis_name="s")

    def make_gather_feat(s):
        @functools.partial(
            pl.kernel,
            out_type=(_f32(ES, D), _f32(ES, D)),
            mesh=mesh,
            scratch_types=[
                pltpu.VMEM((CH,), jnp.int32),
                pltpu.VMEM((CH, D), jnp.float32),
            ],
        )
        def sc_gather_feat(feat_hbm, src_hbm, dst_hbm, xs_hbm, xd_hbm,
                           idx_v, rows_v):
            wid = lax.axis_index("s") * NC + lax.axis_index("c")
            lbase = wid * PW
            gbase = s * ES + wid * PW
            for idx_hbm, out_hbm in ((src_hbm, xs_hbm), (dst_hbm, xd_hbm)):
                @pl.loop(0, NCH)
                def _(c, idx_hbm=idx_hbm, out_hbm=out_hbm):
                    pltpu.sync_copy(idx_hbm.at[pl.ds(gbase + c * CH, CH)],
                                    idx_v)
                    pltpu.sync_copy(feat_hbm.at[idx_v], rows_v)
                    pltpu.sync_copy(rows_v, out_hbm.at[pl.ds(lbase + c * CH,
                                                             CH)])

        return sc_gather_feat

    def make_gather_z(s):
        @functools.partial(
            pl.kernel,
            out_type=_f32(ES, D),
            mesh=mesh,
            scratch_types=[
                pltpu.VMEM((CH,), jnp.int32),
                pltpu.VMEM((CH, D), jnp.float32),
            ],
        )
        def sc_gather_z(z_hbm, dst_hbm, zd_hbm, idx_v, rows_v):
            wid = lax.axis_index("s") * NC + lax.axis_index("c")
            lbase = wid * PW
            gbase = s * ES + wid * PW

            @pl.loop(0, NCH)
            def _(c):
                pltpu.sync_copy(dst_hbm.at[pl.ds(gbase + c * CH, CH)], idx_v)
                pltpu.sync_copy(z_hbm.at[idx_v], rows_v)
                pltpu.sync_copy(rows_v, zd_hbm.at[pl.ds(lbase + c * CH, CH)])

        return sc_gather_z

    def make_scatter_out(s):
        @functools.partial(
            pl.kernel,
            out_type=(_f32(N, D), _f32(N, D)),
            mesh=mesh,
            scratch_types=[
                pltpu.VMEM((CH,), jnp.int32),
                pltpu.VMEM((CH, D), jnp.float32),
                pltpu.VMEM_SHARED((N, D), jnp.float32),
            ],
        )
        def sc_scatter_out(ev_hbm, dst_hbm, zero_hbm, o0_hbm, o1_hbm,
                           idx_v, ev_v, acc_sh):
            core = lax.axis_index("c")
            sid = lax.axis_index("s")

            @pl.when(sid == 0)
            def _():
                pltpu.sync_copy(zero_hbm, acc_sh)
            plsc.subcore_barrier()
            wid = sid * NC + core
            lbase = wid * PW
            gbase = s * ES + wid * PW

            @pl.loop(0, NCH)
            def _(c):
                pltpu.sync_copy(dst_hbm.at[pl.ds(gbase + c * CH, CH)], idx_v)
                pltpu.sync_copy(ev_hbm.at[pl.ds(lbase + c * CH, CH)], ev_v)
                pltpu.sync_copy(ev_v, acc_sh.at[idx_v], add=True)

            plsc.subcore_barrier()

            @pl.when(sid == 0)
            def _():
                @pl.when(core == 0)
                def _():
                    pltpu.sync_copy(acc_sh, o0_hbm)

                @pl.when(core == 1)
                def _():
                    pltpu.sync_copy(acc_sh, o1_hbm)

        return sc_scatter_out

    return make_gather_feat, make_gather_z, make_scatter_out


def kernel(edge_src, edge_dst, edge_weight_cutoff, edge_attr, node_feat,
           Wk0, Wk1, Wk2, Wlogit, Wv0, Wv1, Wv2, Wout):
    make_gather_feat, make_gather_z, make_scatter_out = _build_sc_kernels()

    wk0 = Wk0 * (1.0 / math.sqrt(DE))
    wk1 = Wk1 * (1.0 / 8.0)
    wk2 = Wk2 * (1.0 / 8.0)
    wv0 = Wv0 * (1.0 / math.sqrt(DE))
    wv1 = Wv1 * (1.0 / 8.0)
    wv2 = Wv2 * (1.0 / 8.0)
    wl2 = jnp.transpose(Wlogit, (1, 2, 0)).reshape(D, H * D)
    wout = Wout * (1.0 / math.sqrt(D))
    cutf = jnp.broadcast_to(edge_weight_cutoff[:, None], (E, D))
    eye = jnp.eye(H, dtype=jnp.float32)
    hsel = jnp.repeat(eye, D, axis=0).reshape(H * D, H)
    zerosD = jnp.zeros((N, D), jnp.float32)

    xs, xd, w = [], [], []
    for s in range(SLICES):
        a, b = make_gather_feat(s)(node_feat, edge_src, edge_dst)
        xs.append(a)
        xd.append(b)
    for s in range(SLICES):
        w.append(_tc_logits(s, edge_attr, xs[s], xd[s], cutf,
                            wk0, wk1, wk2, wl2, hsel))
    zparts = []
    for s in range(SLICES):
        zparts.extend(make_scatter_out(s)(w[s], edge_dst, zerosD))
    z = _tc_addn(zparts)
    parts = []
    for s in range(SLICES):
        zd = make_gather_z(s)(z, edge_dst)
        ev = _tc_edgev(s, edge_attr, xs[s], w[s], zd, wv0, wv1, wv2)
        o0, o1 = make_scatter_out(s)(ev, edge_dst, zerosD)
        parts.extend([o0, o1])
    return _tc_final(parts, wout)


# speedup vs baseline: 4.7646x; 1.0595x over previous
import functools
import math

import jax
import jax.numpy as jnp
from jax import lax
from jax.experimental import pallas as pl
from jax.experimental.pallas import tpu as pltpu
from jax.experimental.pallas import tpu_sc as plsc

N = 10000
E = 160000
D = 128
DE = 16
H = 2
DH = D // H

SLICES = 5
ES = E // SLICES

NC = 2
NS = 16
NW = NC * NS
PW = ES // NW
CH_G = 1000
NCH_G = PW // CH_G
CH = 200
NCH = PW // CH

BE = 4000
GBS = ES // BE
BN = 2000
GN = N // BN


def _f32(*shape):
    return jax.ShapeDtypeStruct(shape, jnp.float32)


def _dot3(x, w):
    xh = x.astype(jnp.bfloat16)
    xl = (x - xh.astype(jnp.float32)).astype(jnp.bfloat16)
    wh = w.astype(jnp.bfloat16)
    wl = (w - wh.astype(jnp.float32)).astype(jnp.bfloat16)
    out = jnp.dot(xh, wh, preferred_element_type=jnp.float32)
    out = out + jnp.dot(xh, wl, preferred_element_type=jnp.float32)
    out = out + jnp.dot(xl, wh, preferred_element_type=jnp.float32)
    return out


def _dot1(x, w):
    return jnp.dot(x.astype(jnp.bfloat16), w.astype(jnp.bfloat16),
                   preferred_element_type=jnp.float32)


def _attr_body(at_ref, out_ref):
    out_ref[...] = at_ref[...].T


def _tc_attr_rm(attrT):
    return pl.pallas_call(
        _attr_body,
        grid=(10,),
        in_specs=[pl.BlockSpec((DE, 16000), lambda i: (0, i))],
        out_specs=pl.BlockSpec((16000, DE), lambda i: (i, 0)),
        out_shape=_f32(E, DE),
    )(attrT)

def _logit_body(attr_ref, xs_ref, xd_ref, cut_ref, wk0, wk1, wk2, wl2, hsel,
                w_ref):
    x = attr_ref[...]
    x = jax.nn.gelu(_dot1(x, wk0[...]))
    x = jax.nn.gelu(_dot1(x, wk1[...]))
    k = _dot1(x, wk2[...])
    kk = xs_ref[...] * k
    a = _dot1(kk, wl2[...])
    xd = xd_ref[...]
    m = jnp.concatenate([xd, xd], axis=1) * a
    logit = jnp.dot(m.astype(jnp.bfloat16), hsel[...].astype(jnp.bfloat16),
                    preferred_element_type=jnp.float32) * (1.0 / D)
    w01 = cut_ref[:, 0:1] * jnp.exp(jnp.clip(logit, -60.0, 60.0))
    w_ref[...] = jnp.concatenate(
        [jnp.broadcast_to(w01[:, 0:1], (w01.shape[0], DH)),
         jnp.broadcast_to(w01[:, 1:2], (w01.shape[0], DH))], axis=1)


def _tc_logits(s, edge_attr, xs, xd, cutf, wk0, wk1, wk2, wl2, hsel):
    base = s * GBS
    return pl.pallas_call(
        _logit_body,
        grid=(GBS,),
        in_specs=[
            pl.BlockSpec((BE, DE), lambda i: (base + i, 0)),
            pl.BlockSpec((BE, D), lambda i: (i, 0)),
            pl.BlockSpec((BE, D), lambda i: (i, 0)),
            pl.BlockSpec((BE, D), lambda i: (base + i, 0)),
            pl.BlockSpec((DE, 64), lambda i: (0, 0)),
            pl.BlockSpec((64, 64), lambda i: (0, 0)),
            pl.BlockSpec((64, D), lambda i: (0, 0)),
            pl.BlockSpec((D, 2 * D), lambda i: (0, 0)),
            pl.BlockSpec((2 * D, H), lambda i: (0, 0)),
        ],
        out_specs=pl.BlockSpec((BE, D), lambda i: (i, 0)),
        out_shape=_f32(ES, D),
    )(edge_attr, xs, xd, cutf, wk0, wk1, wk2, wl2, hsel)


def _edgev_body(attr_ref, xs_ref, w_ref, zd_ref, wv0, wv1, wv2, ev_ref):
    x = attr_ref[...]
    x = jax.nn.gelu(_dot1(x, wv0[...]))
    x = jax.nn.gelu(_dot1(x, wv1[...]))
    v = _dot1(x, wv2[...])
    v = xs_ref[...] * v
    n = v.shape[0]
    w01 = jnp.concatenate([w_ref[:, 0:1], w_ref[:, DH:DH + 1]], axis=1)
    zd01 = jnp.concatenate([zd_ref[:, 0:1], zd_ref[:, DH:DH + 1]], axis=1)
    zd01 = jnp.where(zd01 == 0.0, 1.0, zd01)
    s01 = jnp.sqrt(jnp.maximum(w01 / zd01, 0.0))
    s = jnp.concatenate([jnp.broadcast_to(s01[:, 0:1], (n, DH)),
                         jnp.broadcast_to(s01[:, 1:2], (n, DH))], axis=1)
    ev_ref[...] = v * s


def _tc_edgev(s, edge_attr, xs, w, zd, wv0, wv1, wv2):
    base = s * GBS
    return pl.pallas_call(
        _edgev_body,
        grid=(GBS,),
        in_specs=[
            pl.BlockSpec((BE, DE), lambda i: (base + i, 0)),
            pl.BlockSpec((BE, D), lambda i: (i, 0)),
            pl.BlockSpec((BE, D), lambda i: (i, 0)),
            pl.BlockSpec((BE, D), lambda i: (i, 0)),
            pl.BlockSpec((DE, 64), lambda i: (0, 0)),
            pl.BlockSpec((64, 64), lambda i: (0, 0)),
            pl.BlockSpec((64, D), lambda i: (0, 0)),
        ],
        out_specs=pl.BlockSpec((BE, D), lambda i: (i, 0)),
        out_shape=_f32(ES, D),
    )(edge_attr, xs, w, zd, wv0, wv1, wv2)


def _addn_body(*refs):
    out_ref = refs[-1]
    acc = refs[0][...]
    for p in refs[1:-1]:
        acc = acc + p[...]
    out_ref[...] = acc


def _tc_addn(parts):
    return pl.pallas_call(
        _addn_body,
        grid=(GN,),
        in_specs=[pl.BlockSpec((BN, D), lambda i: (i, 0)) for _ in parts],
        out_specs=pl.BlockSpec((BN, D), lambda i: (i, 0)),
        out_shape=_f32(N, D),
    )(*parts)


def _final_body(*refs):
    parts = refs[:-2]
    wout = refs[-2]
    out_ref = refs[-1]
    acc = parts[0][...]
    for p in parts[1:]:
        acc = acc + p[...]
    out_ref[...] = _dot3(acc, wout[...])


def _tc_final(parts, wout):
    return pl.pallas_call(
        _final_body,
        grid=(GN,),
        in_specs=[pl.BlockSpec((BN, D), lambda i: (i, 0)) for _ in parts]
        + [pl.BlockSpec((D, D), lambda i: (0, 0))],
        out_specs=pl.BlockSpec((BN, D), lambda i: (i, 0)),
        out_shape=_f32(N, D),
    )(*parts, wout)


def _build_sc_kernels():
    mesh = plsc.VectorSubcoreMesh(core_axis_name="c", subcore_axis_name="s")

    def make_gather_feat(s):
        @functools.partial(
            pl.kernel,
            out_type=(_f32(ES, D), _f32(ES, D)),
            mesh=mesh,
            scratch_types=[
                pltpu.VMEM((CH_G,), jnp.int32),
                pltpu.VMEM((CH_G, D), jnp.float32),
            ],
        )
        def sc_gather_feat(feat_hbm, src_hbm, dst_hbm, xs_hbm, xd_hbm,
                           idx_v, rows_v):
            wid = lax.axis_index("s") * NC + lax.axis_index("c")
            lbase = wid * PW
            gbase = s * ES + wid * PW
            for idx_hbm, out_hbm in ((src_hbm, xs_hbm), (dst_hbm, xd_hbm)):
                @pl.loop(0, NCH_G)
                def _(c, idx_hbm=idx_hbm, out_hbm=out_hbm):
                    pltpu.sync_copy(idx_hbm.at[pl.ds(gbase + c * CH_G, CH_G)],
                                    idx_v)
                    pltpu.sync_copy(feat_hbm.at[idx_v], rows_v)
                    pltpu.sync_copy(rows_v, out_hbm.at[pl.ds(lbase + c * CH_G,
                                                             CH_G)])

        return sc_gather_feat

    def make_gather_z(s):
        @functools.partial(
            pl.kernel,
            out_type=_f32(ES, D),
            mesh=mesh,
            scratch_types=[
                pltpu.VMEM((CH_G,), jnp.int32),
                pltpu.VMEM((CH_G, D), jnp.float32),
            ],
        )
        def sc_gather_z(z_hbm, dst_hbm, zd_hbm, idx_v, rows_v):
            wid = lax.axis_index("s") * NC + lax.axis_index("c")
            lbase = wid * PW
            gbase = s * ES + wid * PW

            @pl.loop(0, NCH_G)
            def _(c):
                pltpu.sync_copy(dst_hbm.at[pl.ds(gbase + c * CH_G, CH_G)],
                                idx_v)
                pltpu.sync_copy(z_hbm.at[idx_v], rows_v)
                pltpu.sync_copy(rows_v,
                                zd_hbm.at[pl.ds(lbase + c * CH_G, CH_G)])

        return sc_gather_z

    def make_scatter_out(s):
        @functools.partial(
            pl.kernel,
            out_type=(_f32(N, D), _f32(N, D)),
            mesh=mesh,
            scratch_types=[
                pltpu.VMEM((CH,), jnp.int32),
                pltpu.VMEM((CH, D), jnp.float32),
                pltpu.VMEM_SHARED((N, D), jnp.float32),
            ],
        )
        def sc_scatter_out(ev_hbm, dst_hbm, zero_hbm, o0_hbm, o1_hbm,
                           idx_v, ev_v, acc_sh):
            core = lax.axis_index("c")
            sid = lax.axis_index("s")

            @pl.when(sid == 0)
            def _():
                pltpu.sync_copy(zero_hbm, acc_sh)
            plsc.subcore_barrier()
            wid = sid * NC + core
            lbase = wid * PW
            gbase = s * ES + wid * PW

            @pl.loop(0, NCH)
            def _(c):
                pltpu.sync_copy(dst_hbm.at[pl.ds(gbase + c * CH, CH)], idx_v)
                pltpu.sync_copy(ev_hbm.at[pl.ds(lbase + c * CH, CH)], ev_v)
                pltpu.sync_copy(ev_v, acc_sh.at[idx_v], add=True)

            plsc.subcore_barrier()

            @pl.when(sid == 0)
            def _():
                @pl.when(core == 0)
                def _():
                    pltpu.sync_copy(acc_sh, o0_hbm)

                @pl.when(core == 1)
                def _():
                    pltpu.sync_copy(acc_sh, o1_hbm)

        return sc_scatter_out

    return make_gather_feat, make_gather_z, make_scatter_out


def kernel(edge_src, edge_dst, edge_weight_cutoff, edge_attr, node_feat,
           Wk0, Wk1, Wk2, Wlogit, Wv0, Wv1, Wv2, Wout):
    make_gather_feat, make_gather_z, make_scatter_out = _build_sc_kernels()

    wk0 = Wk0 * (1.0 / math.sqrt(DE))
    wk1 = Wk1 * (1.0 / 8.0)
    wk2 = Wk2 * (1.0 / 8.0)
    wv0 = Wv0 * (1.0 / math.sqrt(DE))
    wv1 = Wv1 * (1.0 / 8.0)
    wv2 = Wv2 * (1.0 / 8.0)
    wl2 = jnp.transpose(Wlogit, (1, 2, 0)).reshape(D, H * D)
    wout = Wout * (1.0 / math.sqrt(D))
    cutf = jnp.broadcast_to(edge_weight_cutoff[:, None], (E, D))
    attr_rm = _tc_attr_rm(edge_attr.T)
    eye = jnp.eye(H, dtype=jnp.float32)
    hsel = jnp.repeat(eye, D, axis=0).reshape(H * D, H)
    zerosD = jnp.zeros((N, D), jnp.float32)

    xs, xd, w = [], [], []
    for s in range(SLICES):
        a, b = make_gather_feat(s)(node_feat, edge_src, edge_dst)
        xs.append(a)
        xd.append(b)
    for s in range(SLICES):
        w.append(_tc_logits(s, attr_rm, xs[s], xd[s], cutf,
                            wk0, wk1, wk2, wl2, hsel))
    zparts = []
    for s in range(SLICES):
        zparts.extend(make_scatter_out(s)(w[s], edge_dst, zerosD))
    z = _tc_addn(zparts)
    parts = []
    for s in range(SLICES):
        zd = make_gather_z(s)(z, edge_dst)
        ev = _tc_edgev(s, attr_rm, xs[s], w[s], zd, wv0, wv1, wv2)
        o0, o1 = make_scatter_out(s)(ev, edge_dst, zerosD)
        parts.extend([o0, o1])
    return _tc_final(parts, wout)


# speedup vs baseline: 4.9433x; 1.0375x over previous
import functools
import math

import jax
import jax.numpy as jnp
from jax import lax
from jax.experimental import pallas as pl
from jax.experimental.pallas import tpu as pltpu
from jax.experimental.pallas import tpu_sc as plsc

N = 10000
E = 160000
D = 128
DE = 16
H = 2
DH = D // H

SLICES = 5
ES = E // SLICES

NC = 2
NS = 16
NW = NC * NS
PW = ES // NW
CH_G = 1000
NCH_G = PW // CH_G
CH = 200
NCH = PW // CH

BE = 4000
GBS = ES // BE
BN = 2000
GN = N // BN


def _f32(*shape):
    return jax.ShapeDtypeStruct(shape, jnp.float32)


def _dot3(x, w):
    xh = x.astype(jnp.bfloat16)
    xl = (x - xh.astype(jnp.float32)).astype(jnp.bfloat16)
    wh = w.astype(jnp.bfloat16)
    wl = (w - wh.astype(jnp.float32)).astype(jnp.bfloat16)
    out = jnp.dot(xh, wh, preferred_element_type=jnp.float32)
    out = out + jnp.dot(xh, wl, preferred_element_type=jnp.float32)
    out = out + jnp.dot(xl, wh, preferred_element_type=jnp.float32)
    return out


def _dot1(x, w):
    return jnp.dot(x.astype(jnp.bfloat16), w.astype(jnp.bfloat16),
                   preferred_element_type=jnp.float32)


def _attr_body(at_ref, out_ref):
    out_ref[...] = at_ref[...].T


def _tc_attr_rm(attrT):
    return pl.pallas_call(
        _attr_body,
        grid=(10,),
        in_specs=[pl.BlockSpec((DE, 16000), lambda i: (0, i))],
        out_specs=pl.BlockSpec((16000, DE), lambda i: (i, 0)),
        out_shape=_f32(E, DE),
    )(attrT)

def _logit_body(attr_ref, xs_ref, xd_ref, cut_ref, wk0, wk1, wk2, wl2, hsel,
                w_ref):
    x = attr_ref[...]
    x = jax.nn.gelu(_dot1(x, wk0[...]))
    x = jax.nn.gelu(_dot1(x, wk1[...]))
    k = _dot1(x, wk2[...])
    kk = xs_ref[...] * k
    a = _dot1(kk, wl2[...])
    xd = xd_ref[...]
    m = jnp.concatenate([xd, xd], axis=1) * a
    logit = jnp.dot(m.astype(jnp.bfloat16), hsel[...].astype(jnp.bfloat16),
                    preferred_element_type=jnp.float32) * (1.0 / D)
    cut = cut_ref[:, 0:1].astype(jnp.float32)
    w01 = cut * jnp.exp(jnp.clip(logit, -60.0, 60.0))
    w_ref[...] = jnp.concatenate(
        [jnp.broadcast_to(w01[:, 0:1], (w01.shape[0], DH)),
         jnp.broadcast_to(w01[:, 1:2], (w01.shape[0], DH))], axis=1)


def _tc_logits(s, edge_attr, xs, xd, cutf, wk0, wk1, wk2, wl2, hsel):
    base = s * GBS
    return pl.pallas_call(
        _logit_body,
        grid=(GBS,),
        in_specs=[
            pl.BlockSpec((BE, DE), lambda i: (base + i, 0)),
            pl.BlockSpec((BE, D), lambda i: (i, 0)),
            pl.BlockSpec((BE, D), lambda i: (i, 0)),
            pl.BlockSpec((BE, D), lambda i: (base + i, 0)),
            pl.BlockSpec((DE, 64), lambda i: (0, 0)),
            pl.BlockSpec((64, 64), lambda i: (0, 0)),
            pl.BlockSpec((64, D), lambda i: (0, 0)),
            pl.BlockSpec((D, 2 * D), lambda i: (0, 0)),
            pl.BlockSpec((2 * D, H), lambda i: (0, 0)),
        ],
        out_specs=pl.BlockSpec((BE, D), lambda i: (i, 0)),
        out_shape=_f32(ES, D),
    )(edge_attr, xs, xd, cutf, wk0, wk1, wk2, wl2, hsel)


def _edgev_body(attr_ref, xs_ref, w_ref, zd_ref, wv0, wv1, wv2, ev_ref):
    x = attr_ref[...]
    x = jax.nn.gelu(_dot1(x, wv0[...]))
    x = jax.nn.gelu(_dot1(x, wv1[...]))
    v = _dot1(x, wv2[...])
    v = xs_ref[...] * v
    n = v.shape[0]
    w01 = jnp.concatenate([w_ref[:, 0:1], w_ref[:, DH:DH + 1]], axis=1)
    zd01 = jnp.concatenate([zd_ref[:, 0:1], zd_ref[:, DH:DH + 1]], axis=1)
    zd01 = jnp.where(zd01 == 0.0, 1.0, zd01)
    s01 = jnp.sqrt(jnp.maximum(w01 / zd01, 0.0))
    s = jnp.concatenate([jnp.broadcast_to(s01[:, 0:1], (n, DH)),
                         jnp.broadcast_to(s01[:, 1:2], (n, DH))], axis=1)
    ev_ref[...] = v * s


def _tc_edgev(s, edge_attr, xs, w, zd, wv0, wv1, wv2):
    base = s * GBS
    return pl.pallas_call(
        _edgev_body,
        grid=(GBS,),
        in_specs=[
            pl.BlockSpec((BE, DE), lambda i: (base + i, 0)),
            pl.BlockSpec((BE, D), lambda i: (i, 0)),
            pl.BlockSpec((BE, D), lambda i: (i, 0)),
            pl.BlockSpec((BE, D), lambda i: (i, 0)),
            pl.BlockSpec((DE, 64), lambda i: (0, 0)),
            pl.BlockSpec((64, 64), lambda i: (0, 0)),
            pl.BlockSpec((64, D), lambda i: (0, 0)),
        ],
        out_specs=pl.BlockSpec((BE, D), lambda i: (i, 0)),
        out_shape=_f32(ES, D),
    )(edge_attr, xs, w, zd, wv0, wv1, wv2)


def _addn_body(*refs):
    out_ref = refs[-1]
    acc = refs[0][...]
    for p in refs[1:-1]:
        acc = acc + p[...]
    out_ref[...] = acc


def _tc_addn(parts):
    return pl.pallas_call(
        _addn_body,
        grid=(GN,),
        in_specs=[pl.BlockSpec((BN, D), lambda i: (i, 0)) for _ in parts],
        out_specs=pl.BlockSpec((BN, D), lambda i: (i, 0)),
        out_shape=_f32(N, D),
    )(*parts)


def _final_body(*refs):
    parts = refs[:-2]
    wout = refs[-2]
    out_ref = refs[-1]
    acc = parts[0][...]
    for p in parts[1:]:
        acc = acc + p[...]
    out_ref[...] = _dot3(acc, wout[...])


def _tc_final(parts, wout):
    return pl.pallas_call(
        _final_body,
        grid=(GN,),
        in_specs=[pl.BlockSpec((BN, D), lambda i: (i, 0)) for _ in parts]
        + [pl.BlockSpec((D, D), lambda i: (0, 0))],
        out_specs=pl.BlockSpec((BN, D), lambda i: (i, 0)),
        out_shape=_f32(N, D),
    )(*parts, wout)


def _build_sc_kernels():
    mesh = plsc.VectorSubcoreMesh(core_axis_name="c", subcore_axis_name="s")

    def make_gather_feat(s):
        @functools.partial(
            pl.kernel,
            out_type=(_f32(ES, D), _f32(ES, D)),
            mesh=mesh,
            scratch_types=[
                pltpu.VMEM((CH_G,), jnp.int32),
                pltpu.VMEM((CH_G, D), jnp.float32),
            ],
        )
        def sc_gather_feat(feat_hbm, src_hbm, dst_hbm, xs_hbm, xd_hbm,
                           idx_v, rows_v):
            wid = lax.axis_index("s") * NC + lax.axis_index("c")
            lbase = wid * PW
            gbase = s * ES + wid * PW
            for idx_hbm, out_hbm in ((src_hbm, xs_hbm), (dst_hbm, xd_hbm)):
                @pl.loop(0, NCH_G)
                def _(c, idx_hbm=idx_hbm, out_hbm=out_hbm):
                    pltpu.sync_copy(idx_hbm.at[pl.ds(gbase + c * CH_G, CH_G)],
                                    idx_v)
                    pltpu.sync_copy(feat_hbm.at[idx_v], rows_v)
                    pltpu.sync_copy(rows_v, out_hbm.at[pl.ds(lbase + c * CH_G,
                                                             CH_G)])

        return sc_gather_feat

    def make_gather_z(s):
        @functools.partial(
            pl.kernel,
            out_type=_f32(ES, D),
            mesh=mesh,
            scratch_types=[
                pltpu.VMEM((CH_G,), jnp.int32),
                pltpu.VMEM((CH_G, D), jnp.float32),
            ],
        )
        def sc_gather_z(z_hbm, dst_hbm, zd_hbm, idx_v, rows_v):
            wid = lax.axis_index("s") * NC + lax.axis_index("c")
            lbase = wid * PW
            gbase = s * ES + wid * PW

            @pl.loop(0, NCH_G)
            def _(c):
                pltpu.sync_copy(dst_hbm.at[pl.ds(gbase + c * CH_G, CH_G)],
                                idx_v)
                pltpu.sync_copy(z_hbm.at[idx_v], rows_v)
                pltpu.sync_copy(rows_v,
                                zd_hbm.at[pl.ds(lbase + c * CH_G, CH_G)])

        return sc_gather_z

    def make_scatter_out(s):
        @functools.partial(
            pl.kernel,
            out_type=(_f32(N, D), _f32(N, D)),
            mesh=mesh,
            scratch_types=[
                pltpu.VMEM((CH,), jnp.int32),
                pltpu.VMEM((CH, D), jnp.float32),
                pltpu.VMEM_SHARED((N, D), jnp.float32),
            ],
        )
        def sc_scatter_out(ev_hbm, dst_hbm, zero_hbm, o0_hbm, o1_hbm,
                           idx_v, ev_v, acc_sh):
            core = lax.axis_index("c")
            sid = lax.axis_index("s")

            @pl.when(sid == 0)
            def _():
                pltpu.sync_copy(zero_hbm, acc_sh)
            plsc.subcore_barrier()
            wid = sid * NC + core
            lbase = wid * PW
            gbase = s * ES + wid * PW

            @pl.loop(0, NCH)
            def _(c):
                pltpu.sync_copy(dst_hbm.at[pl.ds(gbase + c * CH, CH)], idx_v)
                pltpu.sync_copy(ev_hbm.at[pl.ds(lbase + c * CH, CH)], ev_v)
                pltpu.sync_copy(ev_v, acc_sh.at[idx_v], add=True)

            plsc.subcore_barrier()

            @pl.when(sid == 0)
            def _():
                @pl.when(core == 0)
                def _():
                    pltpu.sync_copy(acc_sh, o0_hbm)

                @pl.when(core == 1)
                def _():
                    pltpu.sync_copy(acc_sh, o1_hbm)

        return sc_scatter_out

    return make_gather_feat, make_gather_z, make_scatter_out


def kernel(edge_src, edge_dst, edge_weight_cutoff, edge_attr, node_feat,
           Wk0, Wk1, Wk2, Wlogit, Wv0, Wv1, Wv2, Wout):
    make_gather_feat, make_gather_z, make_scatter_out = _build_sc_kernels()

    wk0 = Wk0 * (1.0 / math.sqrt(DE))
    wk1 = Wk1 * (1.0 / 8.0)
    wk2 = Wk2 * (1.0 / 8.0)
    wv0 = Wv0 * (1.0 / math.sqrt(DE))
    wv1 = Wv1 * (1.0 / 8.0)
    wv2 = Wv2 * (1.0 / 8.0)
    wl2 = jnp.transpose(Wlogit, (1, 2, 0)).reshape(D, H * D)
    wout = Wout * (1.0 / math.sqrt(D))
    cutf = jnp.broadcast_to(
        edge_weight_cutoff.astype(jnp.bfloat16)[:, None], (E, D))
    attr_rm = _tc_attr_rm(edge_attr.T)
    eye = jnp.eye(H, dtype=jnp.float32)
    hsel = jnp.repeat(eye, D, axis=0).reshape(H * D, H)
    zerosD = jnp.zeros((N, D), jnp.float32)

    xs, xd, w = [], [], []
    for s in range(SLICES):
        a, b = make_gather_feat(s)(node_feat, edge_src, edge_dst)
        xs.append(a)
        xd.append(b)
    for s in range(SLICES):
        w.append(_tc_logits(s, attr_rm, xs[s], xd[s], cutf,
                            wk0, wk1, wk2, wl2, hsel))
    zparts = []
    for s in range(SLICES):
        zparts.extend(make_scatter_out(s)(w[s], edge_dst, zerosD))
    z = _tc_addn(zparts)
    parts = []
    for s in range(SLICES):
        zd = make_gather_z(s)(z, edge_dst)
        ev = _tc_edgev(s, attr_rm, xs[s], w[s], zd, wv0, wv1, wv2)
        o0, o1 = make_scatter_out(s)(ev, edge_dst, zerosD)
        parts.extend([o0, o1])
    return _tc_final(parts, wout)


# speedup vs baseline: 5.2460x; 1.0612x over previous
import functools
import math

import jax
import jax.numpy as jnp
from jax import lax
from jax.experimental import pallas as pl
from jax.experimental.pallas import tpu as pltpu
from jax.experimental.pallas import tpu_sc as plsc

N = 10000
E = 160000
D = 128
DE = 16
H = 2
DH = D // H

SLICES = 5
ES = E // SLICES

NC = 2
NS = 16
NW = NC * NS
PW = ES // NW
CH_G = 1000
NCH_G = PW // CH_G
CH = 200
NCH = PW // CH

BE = 3200
GBS = ES // BE
BN = 2000
GN = N // BN


def _f32(*shape):
    return jax.ShapeDtypeStruct(shape, jnp.float32)


def _dot3(x, w):
    xh = x.astype(jnp.bfloat16)
    xl = (x - xh.astype(jnp.float32)).astype(jnp.bfloat16)
    wh = w.astype(jnp.bfloat16)
    wl = (w - wh.astype(jnp.float32)).astype(jnp.bfloat16)
    out = jnp.dot(xh, wh, preferred_element_type=jnp.float32)
    out = out + jnp.dot(xh, wl, preferred_element_type=jnp.float32)
    out = out + jnp.dot(xl, wh, preferred_element_type=jnp.float32)
    return out


def _dot1(x, w):
    return jnp.dot(x.astype(jnp.bfloat16), w.astype(jnp.bfloat16),
                   preferred_element_type=jnp.float32)


def _dot1t(xT, w):
    return jax.lax.dot_general(
        xT.astype(jnp.bfloat16), w.astype(jnp.bfloat16),
        (((0,), (0,)), ((), ())), preferred_element_type=jnp.float32)


def _logit_body(attr_ref, xs_ref, xd_ref, cut_ref, wk0, wk1, wk2, wl2, hsel,
                w_ref):
    x = jax.nn.gelu(_dot1t(attr_ref[...], wk0[...]))
    x = jax.nn.gelu(_dot1(x, wk1[...]))
    k = _dot1(x, wk2[...])
    kk = xs_ref[...] * k
    a = _dot1(kk, wl2[...])
    xd = xd_ref[...]
    m = jnp.concatenate([xd, xd], axis=1) * a
    logit = jnp.dot(m.astype(jnp.bfloat16), hsel[...].astype(jnp.bfloat16),
                    preferred_element_type=jnp.float32) * (1.0 / D)
    cut = cut_ref[:, 0:1].astype(jnp.float32)
    w01 = cut * jnp.exp(jnp.clip(logit, -60.0, 60.0))
    w_ref[...] = jnp.concatenate(
        [jnp.broadcast_to(w01[:, 0:1], (w01.shape[0], DH)),
         jnp.broadcast_to(w01[:, 1:2], (w01.shape[0], DH))], axis=1)


def _tc_logits(s, edge_attr, xs, xd, cutf, wk0, wk1, wk2, wl2, hsel):
    base = s * GBS
    return pl.pallas_call(
        _logit_body,
        grid=(GBS,),
        in_specs=[
            pl.BlockSpec((DE, BE), lambda i: (0, base + i)),
            pl.BlockSpec((BE, D), lambda i: (i, 0)),
            pl.BlockSpec((BE, D), lambda i: (i, 0)),
            pl.BlockSpec((BE, D), lambda i: (base + i, 0)),
            pl.BlockSpec((DE, 64), lambda i: (0, 0)),
            pl.BlockSpec((64, 64), lambda i: (0, 0)),
            pl.BlockSpec((64, D), lambda i: (0, 0)),
            pl.BlockSpec((D, 2 * D), lambda i: (0, 0)),
            pl.BlockSpec((2 * D, H), lambda i: (0, 0)),
        ],
        out_specs=pl.BlockSpec((BE, D), lambda i: (i, 0)),
        out_shape=_f32(ES, D),
    )(edge_attr, xs, xd, cutf, wk0, wk1, wk2, wl2, hsel)


def _edgev_body(attr_ref, xs_ref, w_ref, zd_ref, wv0, wv1, wv2, ev_ref):
    x = jax.nn.gelu(_dot1t(attr_ref[...], wv0[...]))
    x = jax.nn.gelu(_dot1(x, wv1[...]))
    v = _dot1(x, wv2[...])
    v = xs_ref[...] * v
    n = v.shape[0]
    w01 = jnp.concatenate([w_ref[:, 0:1], w_ref[:, DH:DH + 1]], axis=1)
    zd01 = jnp.concatenate([zd_ref[:, 0:1], zd_ref[:, DH:DH + 1]], axis=1)
    zd01 = jnp.where(zd01 == 0.0, 1.0, zd01)
    s01 = jnp.sqrt(jnp.maximum(w01 / zd01, 0.0))
    s = jnp.concatenate([jnp.broadcast_to(s01[:, 0:1], (n, DH)),
                         jnp.broadcast_to(s01[:, 1:2], (n, DH))], axis=1)
    ev_ref[...] = v * s


def _tc_edgev(s, edge_attr, xs, w, zd, wv0, wv1, wv2):
    base = s * GBS
    return pl.pallas_call(
        _edgev_body,
        grid=(GBS,),
        in_specs=[
            pl.BlockSpec((DE, BE), lambda i: (0, base + i)),
            pl.BlockSpec((BE, D), lambda i: (i, 0)),
            pl.BlockSpec((BE, D), lambda i: (i, 0)),
            pl.BlockSpec((BE, D), lambda i: (i, 0)),
            pl.BlockSpec((DE, 64), lambda i: (0, 0)),
            pl.BlockSpec((64, 64), lambda i: (0, 0)),
            pl.BlockSpec((64, D), lambda i: (0, 0)),
        ],
        out_specs=pl.BlockSpec((BE, D), lambda i: (i, 0)),
        out_shape=_f32(ES, D),
    )(edge_attr, xs, w, zd, wv0, wv1, wv2)


def _addn_body(*refs):
    out_ref = refs[-1]
    acc = refs[0][...]
    for p in refs[1:-1]:
        acc = acc + p[...]
    out_ref[...] = acc


def _tc_addn(parts):
    return pl.pallas_call(
        _addn_body,
        grid=(GN,),
        in_specs=[pl.BlockSpec((BN, D), lambda i: (i, 0)) for _ in parts],
        out_specs=pl.BlockSpec((BN, D), lambda i: (i, 0)),
        out_shape=_f32(N, D),
    )(*parts)


def _final_body(*refs):
    parts = refs[:-2]
    wout = refs[-2]
    out_ref = refs[-1]
    acc = parts[0][...]
    for p in parts[1:]:
        acc = acc + p[...]
    out_ref[...] = _dot3(acc, wout[...])


def _tc_final(parts, wout):
    return pl.pallas_call(
        _final_body,
        grid=(GN,),
        in_specs=[pl.BlockSpec((BN, D), lambda i: (i, 0)) for _ in parts]
        + [pl.BlockSpec((D, D), lambda i: (0, 0))],
        out_specs=pl.BlockSpec((BN, D), lambda i: (i, 0)),
        out_shape=_f32(N, D),
    )(*parts, wout)


def _build_sc_kernels():
    mesh = plsc.VectorSubcoreMesh(core_axis_name="c", subcore_axis_name="s")

    def make_gather_feat(s):
        @functools.partial(
            pl.kernel,
            out_type=(_f32(ES, D), _f32(ES, D)),
            mesh=mesh,
            scratch_types=[
                pltpu.VMEM((CH_G,), jnp.int32),
                pltpu.VMEM((CH_G, D), jnp.float32),
            ],
        )
        def sc_gather_feat(feat_hbm, src_hbm, dst_hbm, xs_hbm, xd_hbm,
                           idx_v, rows_v):
            wid = lax.axis_index("s") * NC + lax.axis_index("c")
            lbase = wid * PW
            gbase = s * ES + wid * PW
            for idx_hbm, out_hbm in ((src_hbm, xs_hbm), (dst_hbm, xd_hbm)):
                @pl.loop(0, NCH_G)
                def _(c, idx_hbm=idx_hbm, out_hbm=out_hbm):
                    pltpu.sync_copy(idx_hbm.at[pl.ds(gbase + c * CH_G, CH_G)],
                                    idx_v)
                    pltpu.sync_copy(feat_hbm.at[idx_v], rows_v)
                    pltpu.sync_copy(rows_v, out_hbm.at[pl.ds(lbase + c * CH_G,
                                                             CH_G)])

        return sc_gather_feat

    def make_gather_z(s):
        @functools.partial(
            pl.kernel,
            out_type=_f32(ES, D),
            mesh=mesh,
            scratch_types=[
                pltpu.VMEM((CH_G,), jnp.int32),
                pltpu.VMEM((CH_G, D), jnp.float32),
            ],
        )
        def sc_gather_z(z_hbm, dst_hbm, zd_hbm, idx_v, rows_v):
            wid = lax.axis_index("s") * NC + lax.axis_index("c")
            lbase = wid * PW
            gbase = s * ES + wid * PW

            @pl.loop(0, NCH_G)
            def _(c):
                pltpu.sync_copy(dst_hbm.at[pl.ds(gbase + c * CH_G, CH_G)],
                                idx_v)
                pltpu.sync_copy(z_hbm.at[idx_v], rows_v)
                pltpu.sync_copy(rows_v,
                                zd_hbm.at[pl.ds(lbase + c * CH_G, CH_G)])

        return sc_gather_z

    def make_scatter_out(s):
        @functools.partial(
            pl.kernel,
            out_type=(_f32(N, D), _f32(N, D)),
            mesh=mesh,
            scratch_types=[
                pltpu.VMEM((CH,), jnp.int32),
                pltpu.VMEM((CH, D), jnp.float32),
                pltpu.VMEM_SHARED((N, D), jnp.float32),
            ],
        )
        def sc_scatter_out(ev_hbm, dst_hbm, zero_hbm, o0_hbm, o1_hbm,
                           idx_v, ev_v, acc_sh):
            core = lax.axis_index("c")
            sid = lax.axis_index("s")

            @pl.when(sid == 0)
            def _():
                pltpu.sync_copy(zero_hbm, acc_sh)
            plsc.subcore_barrier()
            wid = sid * NC + core
            lbase = wid * PW
            gbase = s * ES + wid * PW

            @pl.loop(0, NCH)
            def _(c):
                pltpu.sync_copy(dst_hbm.at[pl.ds(gbase + c * CH, CH)], idx_v)
                pltpu.sync_copy(ev_hbm.at[pl.ds(lbase + c * CH, CH)], ev_v)
                pltpu.sync_copy(ev_v, acc_sh.at[idx_v], add=True)

            plsc.subcore_barrier()

            @pl.when(sid == 0)
            def _():
                @pl.when(core == 0)
                def _():
                    pltpu.sync_copy(acc_sh, o0_hbm)

                @pl.when(core == 1)
                def _():
                    pltpu.sync_copy(acc_sh, o1_hbm)

        return sc_scatter_out

    return make_gather_feat, make_gather_z, make_scatter_out


def kernel(edge_src, edge_dst, edge_weight_cutoff, edge_attr, node_feat,
           Wk0, Wk1, Wk2, Wlogit, Wv0, Wv1, Wv2, Wout):
    make_gather_feat, make_gather_z, make_scatter_out = _build_sc_kernels()

    wk0 = Wk0 * (1.0 / math.sqrt(DE))
    wk1 = Wk1 * (1.0 / 8.0)
    wk2 = Wk2 * (1.0 / 8.0)
    wv0 = Wv0 * (1.0 / math.sqrt(DE))
    wv1 = Wv1 * (1.0 / 8.0)
    wv2 = Wv2 * (1.0 / 8.0)
    wl2 = jnp.transpose(Wlogit, (1, 2, 0)).reshape(D, H * D)
    wout = Wout * (1.0 / math.sqrt(D))
    cutf = jnp.broadcast_to(
        edge_weight_cutoff.astype(jnp.bfloat16)[:, None], (E, D))
    attrT = edge_attr.T
    eye = jnp.eye(H, dtype=jnp.float32)
    hsel = jnp.repeat(eye, D, axis=0).reshape(H * D, H)
    zerosD = jnp.zeros((N, D), jnp.float32)

    xs, xd, w = [], [], []
    for s in range(SLICES):
        a, b = make_gather_feat(s)(node_feat, edge_src, edge_dst)
        xs.append(a)
        xd.append(b)
    for s in range(SLICES):
        w.append(_tc_logits(s, attrT, xs[s], xd[s], cutf,
                            wk0, wk1, wk2, wl2, hsel))
    zparts = []
    for s in range(SLICES):
        zparts.extend(make_scatter_out(s)(w[s], edge_dst, zerosD))
    z = _tc_addn(zparts)
    parts = []
    for s in range(SLICES):
        zd = make_gather_z(s)(z, edge_dst)
        ev = _tc_edgev(s, attrT, xs[s], w[s], zd, wv0, wv1, wv2)
        o0, o1 = make_scatter_out(s)(ev, edge_dst, zerosD)
        parts.extend([o0, o1])
    return _tc_final(parts, wout)
